# baseline (speedup 1.0000x reference)
import functools
import math

import jax
import jax.numpy as jnp
from jax import lax
from jax.experimental import pallas as pl
from jax.experimental.pallas import tpu as pltpu
from jax.experimental.pallas import tpu_sc as plsc

H = 16
EPS = 1e-5
THETA = 10000.0
SCALE = 1.0
F32 = jnp.float32
NEG = -1e30


BF16 = jnp.bfloat16


def _dot(a, b):
    return jnp.dot(a.astype(BF16), b.astype(BF16),
                   preferred_element_type=F32)

def _rms(x, g):
    return x * jax.lax.rsqrt(jnp.mean(x * x, axis=-1, keepdims=True) + EPS) * g


def _rmsnorm_mm_kern(x_ref, g_ref, w_ref, o_ref):
    h = _rms(x_ref[...], g_ref[...])
    o_ref[...] = _dot(h, w_ref[...]).astype(o_ref.dtype)


def rmsnorm_mm(x, g, w, bm, out_dtype=F32):
    S, D = x.shape
    N = w.shape[1]
    return pl.pallas_call(
        _rmsnorm_mm_kern,
        grid=(S // bm,),
        in_specs=[
            pl.BlockSpec((bm, D), lambda i: (i, 0)),
            pl.BlockSpec((1, D), lambda i: (0, 0)),
            pl.BlockSpec((D, N), lambda i: (0, 0)),
        ],
        out_specs=pl.BlockSpec((bm, N), lambda i: (i, 0)),
        out_shape=jax.ShapeDtypeStruct((S, N), out_dtype),
    )(x, g.reshape(1, D), w)


def _ropek_kern(kt_ref, ct_ref, st_ref, o_ref):
    kt = kt_ref[0].astype(F32)
    half = kt.shape[0] // 2
    ct = ct_ref[...]
    st = st_ref[...]
    o_ref[0] = jnp.concatenate(
        [kt[:half] * ct + kt[half:] * st,
         -kt[:half] * st + kt[half:] * ct], axis=0).astype(o_ref.dtype)


def rope_k(kht, cosT, sinT):
    Hh, A, S = kht.shape
    half = A // 2
    return pl.pallas_call(
        _ropek_kern,
        grid=(Hh,),
        in_specs=[
            pl.BlockSpec((1, A, S), lambda h: (h, 0, 0)),
            pl.BlockSpec((half, S), lambda h: (0, 0)),
            pl.BlockSpec((half, S), lambda h: (0, 0)),
        ],
        out_specs=pl.BlockSpec((1, A, S), lambda h: (h, 0, 0)),
        out_shape=jax.ShapeDtypeStruct((Hh, A, S), BF16),
    )(kht, cosT, sinT)


def _mask_kern(dq_ref, o_ref, *, bq, S):
    i = pl.program_id(0)
    qpos = i * bq + jax.lax.broadcasted_iota(jnp.int32, (bq, 1), 0)
    kpos = jax.lax.broadcasted_iota(jnp.int32, (bq, S), 1)
    ok = (kpos <= qpos) & (kpos >= dq_ref[...])
    o_ref[...] = jnp.where(ok, 0.0, NEG).astype(o_ref.dtype)


def build_mask(doc_start_col, S, bq):
    kern = functools.partial(_mask_kern, bq=bq, S=S)
    return pl.pallas_call(
        kern,
        grid=(S // bq,),
        in_specs=[pl.BlockSpec((bq, 1), lambda i: (i, 0))],
        out_specs=pl.BlockSpec((bq, S), lambda i: (i, 0)),
        out_shape=jax.ShapeDtypeStruct((S, S), F32),
    )(doc_start_col)


def _attn_kern(q_ref, kt_ref, v_ref, cq_ref, sq_ref, mask_ref, o_ref, *, A):
    half = A // 2
    q = q_ref[0].astype(F32)
    q = jnp.concatenate([q[:, :half] * cq_ref[...] + q[:, half:] * sq_ref[...],
                         -q[:, :half] * sq_ref[...] + q[:, half:] * cq_ref[...]],
                        axis=-1) * (1.0 / math.sqrt(A))
    s = jnp.dot(q.astype(BF16), kt_ref[0], preferred_element_type=F32)
    s = s + mask_ref[...]
    m = jnp.max(s, axis=-1, keepdims=True)
    p = jnp.exp(s - m)
    l = jnp.sum(p, axis=-1, keepdims=True)
    o = jnp.dot(p.astype(BF16), v_ref[0], preferred_element_type=F32)
    o_ref[0] = (o / l).astype(o_ref.dtype)


def attention(qkv, cos, sin, cosT, sinT, mask, bq):
    S = qkv.shape[0]
    D = qkv.shape[1] // 3
    A = D // H
    half = A // 2
    nq = S // bq
    qh = qkv[:, :D].reshape(S, H, A).transpose(1, 0, 2)
    kht = rope_k(qkv[:, D:2 * D].reshape(S, H, A).transpose(1, 2, 0),
                 cosT, sinT)
    vh = qkv[:, 2 * D:].reshape(S, H, A).transpose(1, 0, 2)
    kern = functools.partial(_attn_kern, A=A)
    out = pl.pallas_call(
        kern,
        grid=(nq, H),
        in_specs=[
            pl.BlockSpec((1, bq, A), lambda i, h: (h, i, 0)),
            pl.BlockSpec((1, A, S), lambda i, h: (h, 0, 0)),
            pl.BlockSpec((1, S, A), lambda i, h: (h, 0, 0)),
            pl.BlockSpec((bq, half), lambda i, h: (i, 0)),
            pl.BlockSpec((bq, half), lambda i, h: (i, 0)),
            pl.BlockSpec((bq, S), lambda i, h: (i, 0)),
        ],
        out_specs=pl.BlockSpec((1, bq, A), lambda i, h: (h, i, 0)),
        out_shape=jax.ShapeDtypeStruct((H, S, A), BF16),
    )(qh, kht, vh, cos, sin, mask)
    return out.transpose(1, 0, 2).reshape(S, D)


def _mm_add_kern(a_ref, w_ref, r_ref, o_ref):
    o_ref[...] = _dot(a_ref[...], w_ref[...]) + r_ref[...]


def mm_add(a, w, res, bm):
    S, K = a.shape
    N = w.shape[1]
    return pl.pallas_call(
        _mm_add_kern,
        grid=(S // bm,),
        in_specs=[
            pl.BlockSpec((bm, K), lambda i: (i, 0)),
            pl.BlockSpec((K, N), lambda i: (0, 0)),
            pl.BlockSpec((bm, N), lambda i: (i, 0)),
        ],
        out_specs=pl.BlockSpec((bm, N), lambda i: (i, 0)),
        out_shape=jax.ShapeDtypeStruct((S, N), F32),
    )(a, w, res)


def _ffn_kern(x_ref, g_ref, wu_ref, wd_ref, y_ref, hf_ref):
    x = x_ref[...]
    h = _rms(x, g_ref[...])
    hf_ref[...] = h
    u = _dot(h, wu_ref[...])
    F = wd_ref.shape[0]
    a1 = u[:, :F]
    a2 = u[:, F:]
    gated = a1 * jax.nn.sigmoid(a1) * a2
    y_ref[...] = _dot(gated, wd_ref[...]) + x


def ffn(x2, g, wup, wdn, bm):
    S, D = x2.shape
    N = wup.shape[1]
    F = wdn.shape[0]
    return pl.pallas_call(
        _ffn_kern,
        grid=(S // bm,),
        in_specs=[
            pl.BlockSpec((bm, D), lambda i: (i, 0)),
            pl.BlockSpec((1, D), lambda i: (0, 0)),
            pl.BlockSpec((D, N), lambda i: (0, 0)),
            pl.BlockSpec((F, D), lambda i: (0, 0)),
        ],
        out_specs=[
            pl.BlockSpec((bm, D), lambda i: (i, 0)),
            pl.BlockSpec((bm, D), lambda i: (i, 0)),
        ],
        out_shape=[
            jax.ShapeDtypeStruct((S, D), F32),
            jax.ShapeDtypeStruct((S, D), F32),
        ],
    )(x2, g.reshape(1, D), wup, wdn)


def _mm_kern(a_ref, w_ref, o_ref):
    o_ref[...] = _dot(a_ref[...], w_ref[...])


def mm_plain(a, w):
    S, K = a.shape
    N = w.shape[1]
    return pl.pallas_call(
        _mm_kern,
        in_specs=[pl.BlockSpec((S, K), lambda: (0, 0)),
                  pl.BlockSpec((K, N), lambda: (0, 0))],
        out_specs=pl.BlockSpec((S, N), lambda: (0, 0)),
        out_shape=jax.ShapeDtypeStruct((S, N), F32),
    )(a, w)


def _mm_add_norm_kern(a_ref, w_ref, r_ref, g_ref, o_ref, hf_ref):
    x2 = _dot(a_ref[...], w_ref[...]) + r_ref[...]
    o_ref[...] = x2
    hf_ref[...] = _rms(x2, g_ref[...]).astype(hf_ref.dtype)


def mm_add_norm(a, w, res, g, bm):
    S, K = a.shape
    N = w.shape[1]
    return pl.pallas_call(
        _mm_add_norm_kern,
        grid=(S // bm,),
        in_specs=[
            pl.BlockSpec((bm, K), lambda i: (i, 0)),
            pl.BlockSpec((K, N), lambda i: (0, 0)),
            pl.BlockSpec((bm, N), lambda i: (i, 0)),
            pl.BlockSpec((1, N), lambda i: (0, 0)),
        ],
        out_specs=[
            pl.BlockSpec((bm, N), lambda i: (i, 0)),
            pl.BlockSpec((bm, N), lambda i: (i, 0)),
        ],
        out_shape=[
            jax.ShapeDtypeStruct((S, N), F32),
            jax.ShapeDtypeStruct((S, N), BF16),
        ],
    )(a, w, res, g.reshape(1, N))


def _rmsnorm_kern(x_ref, g_ref, o_ref):
    o_ref[...] = _rms(x_ref[...], g_ref[...]).astype(o_ref.dtype)


def rmsnorm_only(x, g, bm):
    S, D = x.shape
    return pl.pallas_call(
        _rmsnorm_kern,
        grid=(S // bm,),
        in_specs=[
            pl.BlockSpec((bm, D), lambda i: (i, 0)),
            pl.BlockSpec((1, D), lambda i: (0, 0)),
        ],
        out_specs=pl.BlockSpec((bm, D), lambda i: (i, 0)),
        out_shape=jax.ShapeDtypeStruct((S, D), BF16),
    )(x, g.reshape(1, D))


def _ffn2_kern(h_ref, r_ref, wu_ref, wd_ref, y_ref):
    h = h_ref[...]
    u = _dot(h, wu_ref[...])
    F = wd_ref.shape[0]
    a1 = u[:, :F]
    a2 = u[:, F:]
    gated = a1 * jax.nn.sigmoid(a1) * a2
    y_ref[...] = _dot(gated, wd_ref[...]) + r_ref[...]


def ffn_from_hf(hf, res, wup, wdn, bm):
    S, D = hf.shape
    N = wup.shape[1]
    F = wdn.shape[0]
    return pl.pallas_call(
        _ffn2_kern,
        grid=(S // bm,),
        in_specs=[
            pl.BlockSpec((bm, D), lambda i: (i, 0)),
            pl.BlockSpec((bm, D), lambda i: (i, 0)),
            pl.BlockSpec((D, N), lambda i: (0, 0)),
            pl.BlockSpec((F, D), lambda i: (0, 0)),
        ],
        out_specs=pl.BlockSpec((bm, D), lambda i: (i, 0)),
        out_shape=jax.ShapeDtypeStruct((S, D), F32),
    )(hf, res, wup, wdn)


def _make_sc_router(T, LE, NW=32, L=16):
    tpw = T // NW
    nv = tpw // L
    mesh = plsc.VectorSubcoreMesh(core_axis_name="c", subcore_axis_name="s")

    @functools.partial(
        pl.kernel, mesh=mesh,
        compiler_params=pltpu.CompilerParams(needs_layout_passes=False),
        out_type=jax.ShapeDtypeStruct((LE * T,), F32),
        scratch_types=[
            pltpu.VMEM((T * LE,), F32),
            pltpu.VMEM((LE,), F32),
            pltpu.VMEM((tpw,), jnp.int32),
            pltpu.VMEM((tpw,), jnp.int32),
            pltpu.VMEM((tpw,), F32),
            pltpu.VMEM((tpw,), F32),
            pltpu.VMEM((LE, tpw), F32),
        ],
    )
    def sc_router(tv_hbm, idx0_hbm, idx1_hbm, val0_hbm, val1_hbm, rb_hbm,
                  out_hbm, tv_v, rb_v, i0_v, i1_v, v0_v, v1_v, cb_v):
        wid = lax.axis_index("s") * 2 + lax.axis_index("c")
        base = wid * tpw
        pltpu.sync_copy(tv_hbm, tv_v)
        pltpu.sync_copy(rb_hbm, rb_v)
        pltpu.sync_copy(idx0_hbm.at[pl.ds(base, tpw)], i0_v)
        pltpu.sync_copy(idx1_hbm.at[pl.ds(base, tpw)], i1_v)
        pltpu.sync_copy(val0_hbm.at[pl.ds(base, tpw)], v0_v)
        pltpu.sync_copy(val1_hbm.at[pl.ds(base, tpw)], v1_v)
        for i in range(nv):
            sl = pl.ds(i * L, L)
            t_flat = (lax.iota(jnp.int32, L) + (base + i * L)) * LE
            s_k = []
            idxs = []
            for (iv, vv) in ((i0_v, v0_v), (i1_v, v1_v)):
                ik = iv[sl]
                tvk = plsc.load_gather(tv_v, [t_flat + ik])
                rbk = plsc.load_gather(rb_v, [ik])
                xv = vv[sl] + tvk + rbk
                s_k.append(1.0 / (1.0 + jnp.exp(-xv)))
                idxs.append(ik)
            den = s_k[0] + s_k[1]
            c0 = s_k[0] / den * SCALE
            c1 = s_k[1] / den * SCALE
            zero = jnp.zeros((L,), F32)
            for e in range(LE):
                ce = (jnp.where(idxs[0] == e, c0, zero)
                      + jnp.where(idxs[1] == e, c1, zero))
                cb_v[e, sl] = ce
        for e in range(LE):
            pltpu.sync_copy(cb_v.at[e], out_hbm.at[pl.ds(e * T + base, tpw)])

    return sc_router


def sc_router_call(tv, idx, val, rbias):
    T, LE = tv.shape
    fn = _make_sc_router(T, LE)
    out = fn(tv.reshape(T * LE), idx[:, 0].astype(jnp.int32),
             idx[:, 1].astype(jnp.int32), val[:, 0], val[:, 1], rbias)
    return out.reshape(LE, T).T


def _router_kern(tv_ref, idx_ref, val_ref, rb_ref, comb_ref, *, LE, TK):
    T = tv_ref.shape[0]
    tv = tv_ref[:, :LE]
    idx = idx_ref[...]
    val = val_ref[...]
    rb = rb_ref[...]
    lanes = jax.lax.broadcasted_iota(jnp.int32, (T, LE), 1)
    num = jnp.zeros((T, LE), F32)
    den = jnp.zeros((T, 1), F32)
    for kk in range(TK):
        oh = (idx[:, kk:kk + 1] == lanes).astype(F32)
        tvk = jnp.sum(tv * oh, axis=-1, keepdims=True)
        rbk = jnp.sum(rb * oh, axis=-1, keepdims=True)
        s = jax.nn.sigmoid(val[:, kk:kk + 1] + tvk + rbk)
        num = num + oh * s
        den = den + s
    comb_ref[...] = num / den * SCALE


def router(tv, idx, val, rbias):
    T = tv.shape[0]
    LE = rbias.shape[0]
    TK = idx.shape[1]
    kern = functools.partial(_router_kern, LE=LE, TK=TK)
    return pl.pallas_call(
        kern,
        in_specs=[
            pl.BlockSpec(tv.shape, lambda: (0, 0)),
            pl.BlockSpec((T, TK), lambda: (0, 0)),
            pl.BlockSpec((T, TK), lambda: (0, 0)),
            pl.BlockSpec((1, LE), lambda: (0, 0)),
        ],
        out_specs=pl.BlockSpec((T, LE), lambda: (0, 0)),
        out_shape=jax.ShapeDtypeStruct((T, LE), F32),
    )(tv, idx.astype(jnp.int32), val, rbias.reshape(1, LE))


def _moe_kern(hf_ref, w1_ref, w2_ref, w3_ref, comb_ref, res_ref, o_ref, *, LE):
    e = pl.program_id(0)
    h = hf_ref[...]
    lanes = jax.lax.broadcasted_iota(jnp.int32, (1, LE), 1)
    c = jnp.sum(comb_ref[...] * (lanes == e).astype(F32),
                axis=-1, keepdims=True)
    h1 = _dot(h, w1_ref[0])
    h2 = _dot(h, w2_ref[0])
    hh = h1 * jax.nn.sigmoid(h1) * h2
    yo = _dot(hh, w3_ref[0])

    @pl.when(e == 0)
    def _():
        o_ref[...] = res_ref[...]

    o_ref[...] += c * yo


def moe_experts(hf, w1, w2, w3t, comb, res):
    S, D = hf.shape
    LE = w1.shape[0]
    DE = w1.shape[2]
    kern = functools.partial(_moe_kern, LE=LE)
    return pl.pallas_call(
        kern,
        grid=(LE,),
        in_specs=[
            pl.BlockSpec((S, D), lambda e: (0, 0)),
            pl.BlockSpec((1, D, DE), lambda e: (e, 0, 0)),
            pl.BlockSpec((1, D, DE), lambda e: (e, 0, 0)),
            pl.BlockSpec((1, DE, D), lambda e: (e, 0, 0)),
            pl.BlockSpec((S, LE), lambda e: (0, 0)),
            pl.BlockSpec((S, D), lambda e: (0, 0)),
        ],
        out_specs=pl.BlockSpec((S, D), lambda e: (0, 0)),
        out_shape=jax.ShapeDtypeStruct((S, D), F32),
    )(hf, w1, w2, w3t, comb, res)


def _dense_layer(x, rope_doc, wqkv, wo, wup, wdn, g1, g2, bm, bq):
    cos, sin, cosT, sinT, mask = rope_doc
    qkv = rmsnorm_mm(x, g1, wqkv, bm, out_dtype=BF16)
    xa = attention(qkv, cos, sin, cosT, sinT, mask, bq)
    x2 = mm_add(xa, wo, x, bm)
    y, _ = ffn(x2, g2, wup, wdn, bm)
    return y


def _moe_layer(x, rope_doc, idx, val, wqkv, wo, g1, g2, w1, w2, w3t,
               tkeys_pad, rbias, wup, wdn, bm, bq):
    cos, sin, cosT, sinT, mask = rope_doc
    qkv = rmsnorm_mm(x, g1, wqkv, bm, out_dtype=BF16)
    xa = attention(qkv, cos, sin, cosT, sinT, mask, bq)
    x2, hf = mm_add_norm(xa, wo, x, g2, bm)
    tv = mm_plain(hf, tkeys_pad)
    LE = rbias.shape[0]
    comb = sc_router_call(tv[:, :LE], idx, val, rbias)
    y_sh = ffn_from_hf(hf, x2, wup, wdn, bm)
    return moe_experts(hf, w1, w2, w3t, comb, res=y_sh)


def kernel(x, doc, indices, values, dl_attn_w, dl_attn_o_w, dl_ffn_up_w,
           dl_ffn_down_w, dl_attn_norm, dl_ffn_norm, ml_attn_w, ml_attn_o_w,
           ml_attn_norm, ml_ffn_norm, ml_experts, ml_token_keys,
           ml_router_bias, ml_ffn_up_w, ml_ffn_down_w):
    B, S, D = x.shape
    A = D // H
    bm = min(256, S)
    bq = min(512, S)

    inv = (1.0 / THETA) ** (jnp.arange(0, A, 2, dtype=F32) / A)
    fr = jnp.outer(jnp.arange(S, dtype=F32), inv)
    cos, sin = jnp.cos(fr), jnp.sin(fr)
    doc_flat = doc.reshape(S).astype(jnp.int32)
    doc_start = jnp.searchsorted(doc_flat, doc_flat, side="left")
    start_col = doc_start.reshape(S, 1).astype(jnp.int32)
    mask = build_mask(start_col, S, bq)
    rope_doc = (cos, sin, cos.T, sin.T, mask)

    LE = ml_router_bias.shape[1]
    pad_to = max(128, LE)
    xs = x.reshape(S, D)

    dl_attn_w = dl_attn_w.astype(BF16)
    dl_attn_o_w = dl_attn_o_w.astype(BF16)
    dl_ffn_up_w = dl_ffn_up_w.astype(BF16)
    dl_ffn_down_w = dl_ffn_down_w.astype(BF16)
    ml_attn_w = ml_attn_w.astype(BF16)
    ml_attn_o_w = ml_attn_o_w.astype(BF16)
    ml_ffn_up_w = ml_ffn_up_w.astype(BF16)
    ml_ffn_down_w = ml_ffn_down_w.astype(BF16)
    ml_experts_b = ml_experts.astype(BF16)

    xs = _dense_layer(xs, rope_doc, dl_attn_w[0], dl_attn_o_w[0],
                      dl_ffn_up_w[0], dl_ffn_down_w[0], dl_attn_norm[0],
                      dl_ffn_norm[0], bm, bq)

    L = ml_attn_w.shape[0]
    for j in range(L):
        tkeys_pad = jnp.pad(ml_token_keys[j],
                            ((0, 0), (0, pad_to - LE))).astype(BF16)
        w3t = ml_experts_b[j, 2].transpose(0, 2, 1)
        xs = _moe_layer(xs, rope_doc, indices[j], values[j], ml_attn_w[j],
                        ml_attn_o_w[j], ml_attn_norm[j], ml_ffn_norm[j],
                        ml_experts_b[j, 0], ml_experts_b[j, 1], w3t,
                        tkeys_pad, ml_router_bias[j], ml_ffn_up_w[j],
                        ml_ffn_down_w[j], bm, bq)

    xs = _dense_layer(xs, rope_doc, dl_attn_w[1], dl_attn_o_w[1],
                      dl_ffn_up_w[1], dl_ffn_down_w[1], dl_attn_norm[1],
                      dl_ffn_norm[1], bm, bq)
    return xs.reshape(B, S, D)


# speedup vs baseline: 1.0507x; 1.0507x over previous
import functools
import math

import jax
import jax.numpy as jnp
from jax import lax
from jax.experimental import pallas as pl
from jax.experimental.pallas import tpu as pltpu
from jax.experimental.pallas import tpu_sc as plsc

H = 16
EPS = 1e-5
THETA = 10000.0
SCALE = 1.0
F32 = jnp.float32
NEG = -1e30


BF16 = jnp.bfloat16


def _dot(a, b):
    return jnp.dot(a.astype(BF16), b.astype(BF16),
                   preferred_element_type=F32)

def _rms(x, g):
    return x * jax.lax.rsqrt(jnp.mean(x * x, axis=-1, keepdims=True) + EPS) * g


def _rmsnorm_mm_kern(x_ref, g_ref, w_ref, o_ref):
    h = _rms(x_ref[...], g_ref[...])
    o_ref[...] = _dot(h, w_ref[...]).astype(o_ref.dtype)


def rmsnorm_mm(x, g, w, bm, out_dtype=F32):
    S, D = x.shape
    N = w.shape[1]
    return pl.pallas_call(
        _rmsnorm_mm_kern,
        grid=(S // bm,),
        in_specs=[
            pl.BlockSpec((bm, D), lambda i: (i, 0)),
            pl.BlockSpec((1, D), lambda i: (0, 0)),
            pl.BlockSpec((D, N), lambda i: (0, 0)),
        ],
        out_specs=pl.BlockSpec((bm, N), lambda i: (i, 0)),
        out_shape=jax.ShapeDtypeStruct((S, N), out_dtype),
    )(x, g.reshape(1, D), w)


def _ropek_kern(kt_ref, ct_ref, st_ref, o_ref):
    kt = kt_ref[0].astype(F32)
    half = kt.shape[0] // 2
    ct = ct_ref[...]
    st = st_ref[...]
    o_ref[0] = jnp.concatenate(
        [kt[:half] * ct + kt[half:] * st,
         -kt[:half] * st + kt[half:] * ct], axis=0).astype(o_ref.dtype)


def rope_k(kht, cosT, sinT):
    Hh, A, S = kht.shape
    half = A // 2
    return pl.pallas_call(
        _ropek_kern,
        grid=(Hh,),
        in_specs=[
            pl.BlockSpec((1, A, S), lambda h: (h, 0, 0)),
            pl.BlockSpec((half, S), lambda h: (0, 0)),
            pl.BlockSpec((half, S), lambda h: (0, 0)),
        ],
        out_specs=pl.BlockSpec((1, A, S), lambda h: (h, 0, 0)),
        out_shape=jax.ShapeDtypeStruct((Hh, A, S), BF16),
    )(kht, cosT, sinT)


def _mask_kern(dq_ref, o_ref, *, bq, S):
    i = pl.program_id(0)
    qpos = i * bq + jax.lax.broadcasted_iota(jnp.int32, (bq, 1), 0)
    kpos = jax.lax.broadcasted_iota(jnp.int32, (bq, S), 1)
    ok = (kpos <= qpos) & (kpos >= dq_ref[...])
    o_ref[...] = jnp.where(ok, 0.0, NEG).astype(o_ref.dtype)


def build_mask(doc_start_col, S, bq):
    kern = functools.partial(_mask_kern, bq=bq, S=S)
    return pl.pallas_call(
        kern,
        grid=(S // bq,),
        in_specs=[pl.BlockSpec((bq, 1), lambda i: (i, 0))],
        out_specs=pl.BlockSpec((bq, S), lambda i: (i, 0)),
        out_shape=jax.ShapeDtypeStruct((S, S), F32),
    )(doc_start_col)


def _attn_kern(q_ref, kt_ref, v_ref, cq_ref, sq_ref, mask_ref, o_ref, *, A):
    half = A // 2
    q = q_ref[0].astype(F32)
    q = jnp.concatenate([q[:, :half] * cq_ref[...] + q[:, half:] * sq_ref[...],
                         -q[:, :half] * sq_ref[...] + q[:, half:] * cq_ref[...]],
                        axis=-1) * (1.0 / math.sqrt(A))
    s = jnp.dot(q.astype(BF16), kt_ref[0], preferred_element_type=F32)
    s = s + mask_ref[...]
    m = jnp.max(s, axis=-1, keepdims=True)
    p = jnp.exp(s - m)
    l = jnp.sum(p, axis=-1, keepdims=True)
    o = jnp.dot(p.astype(BF16), v_ref[0], preferred_element_type=F32)
    o_ref[0] = (o / l).astype(o_ref.dtype)


def attention(qkv, cos, sin, cosT, sinT, mask, bq):
    S = qkv.shape[0]
    D = qkv.shape[1] // 3
    A = D // H
    half = A // 2
    nq = S // bq
    qh = qkv[:, :D].reshape(S, H, A).transpose(1, 0, 2)
    kht = rope_k(qkv[:, D:2 * D].reshape(S, H, A).transpose(1, 2, 0),
                 cosT, sinT)
    vh = qkv[:, 2 * D:].reshape(S, H, A).transpose(1, 0, 2)
    kern = functools.partial(_attn_kern, A=A)
    out = pl.pallas_call(
        kern,
        grid=(nq, H),
        in_specs=[
            pl.BlockSpec((1, bq, A), lambda i, h: (h, i, 0)),
            pl.BlockSpec((1, A, S), lambda i, h: (h, 0, 0)),
            pl.BlockSpec((1, S, A), lambda i, h: (h, 0, 0)),
            pl.BlockSpec((bq, half), lambda i, h: (i, 0)),
            pl.BlockSpec((bq, half), lambda i, h: (i, 0)),
            pl.BlockSpec((bq, S), lambda i, h: (i, 0)),
        ],
        out_specs=pl.BlockSpec((1, bq, A), lambda i, h: (h, i, 0)),
        out_shape=jax.ShapeDtypeStruct((H, S, A), BF16),
    )(qh, kht, vh, cos, sin, mask)
    return out.transpose(1, 0, 2).reshape(S, D)


def _mm_add_kern(a_ref, w_ref, r_ref, o_ref):
    o_ref[...] = _dot(a_ref[...], w_ref[...]) + r_ref[...]


def mm_add(a, w, res, bm):
    S, K = a.shape
    N = w.shape[1]
    return pl.pallas_call(
        _mm_add_kern,
        grid=(S // bm,),
        in_specs=[
            pl.BlockSpec((bm, K), lambda i: (i, 0)),
            pl.BlockSpec((K, N), lambda i: (0, 0)),
            pl.BlockSpec((bm, N), lambda i: (i, 0)),
        ],
        out_specs=pl.BlockSpec((bm, N), lambda i: (i, 0)),
        out_shape=jax.ShapeDtypeStruct((S, N), F32),
    )(a, w, res)


def _ffn_kern(x_ref, g_ref, wu_ref, wd_ref, y_ref, hf_ref):
    x = x_ref[...]
    h = _rms(x, g_ref[...])
    hf_ref[...] = h
    u = _dot(h, wu_ref[...])
    F = wd_ref.shape[0]
    a1 = u[:, :F]
    a2 = u[:, F:]
    gated = a1 * jax.nn.sigmoid(a1) * a2
    y_ref[...] = _dot(gated, wd_ref[...]) + x


def ffn(x2, g, wup, wdn, bm):
    S, D = x2.shape
    N = wup.shape[1]
    F = wdn.shape[0]
    return pl.pallas_call(
        _ffn_kern,
        grid=(S // bm,),
        in_specs=[
            pl.BlockSpec((bm, D), lambda i: (i, 0)),
            pl.BlockSpec((1, D), lambda i: (0, 0)),
            pl.BlockSpec((D, N), lambda i: (0, 0)),
            pl.BlockSpec((F, D), lambda i: (0, 0)),
        ],
        out_specs=[
            pl.BlockSpec((bm, D), lambda i: (i, 0)),
            pl.BlockSpec((bm, D), lambda i: (i, 0)),
        ],
        out_shape=[
            jax.ShapeDtypeStruct((S, D), F32),
            jax.ShapeDtypeStruct((S, D), F32),
        ],
    )(x2, g.reshape(1, D), wup, wdn)


def _mm_kern(a_ref, w_ref, o_ref):
    o_ref[...] = _dot(a_ref[...], w_ref[...])


def mm_plain(a, w):
    S, K = a.shape
    N = w.shape[1]
    return pl.pallas_call(
        _mm_kern,
        in_specs=[pl.BlockSpec((S, K), lambda: (0, 0)),
                  pl.BlockSpec((K, N), lambda: (0, 0))],
        out_specs=pl.BlockSpec((S, N), lambda: (0, 0)),
        out_shape=jax.ShapeDtypeStruct((S, N), F32),
    )(a, w)


def _mm_add_norm_kern(a_ref, w_ref, r_ref, g_ref, o_ref, hf_ref):
    x2 = _dot(a_ref[...], w_ref[...]) + r_ref[...]
    o_ref[...] = x2
    hf_ref[...] = _rms(x2, g_ref[...]).astype(hf_ref.dtype)


def mm_add_norm(a, w, res, g, bm):
    S, K = a.shape
    N = w.shape[1]
    return pl.pallas_call(
        _mm_add_norm_kern,
        grid=(S // bm,),
        in_specs=[
            pl.BlockSpec((bm, K), lambda i: (i, 0)),
            pl.BlockSpec((K, N), lambda i: (0, 0)),
            pl.BlockSpec((bm, N), lambda i: (i, 0)),
            pl.BlockSpec((1, N), lambda i: (0, 0)),
        ],
        out_specs=[
            pl.BlockSpec((bm, N), lambda i: (i, 0)),
            pl.BlockSpec((bm, N), lambda i: (i, 0)),
        ],
        out_shape=[
            jax.ShapeDtypeStruct((S, N), F32),
            jax.ShapeDtypeStruct((S, N), BF16),
        ],
    )(a, w, res, g.reshape(1, N))


def _rmsnorm_kern(x_ref, g_ref, o_ref):
    o_ref[...] = _rms(x_ref[...], g_ref[...]).astype(o_ref.dtype)


def rmsnorm_only(x, g, bm):
    S, D = x.shape
    return pl.pallas_call(
        _rmsnorm_kern,
        grid=(S // bm,),
        in_specs=[
            pl.BlockSpec((bm, D), lambda i: (i, 0)),
            pl.BlockSpec((1, D), lambda i: (0, 0)),
        ],
        out_specs=pl.BlockSpec((bm, D), lambda i: (i, 0)),
        out_shape=jax.ShapeDtypeStruct((S, D), BF16),
    )(x, g.reshape(1, D))


def _ffn2_kern(h_ref, r_ref, wu_ref, wd_ref, y_ref):
    h = h_ref[...]
    u = _dot(h, wu_ref[...])
    F = wd_ref.shape[0]
    a1 = u[:, :F]
    a2 = u[:, F:]
    gated = a1 * jax.nn.sigmoid(a1) * a2
    y_ref[...] = _dot(gated, wd_ref[...]) + r_ref[...]


def ffn_from_hf(hf, res, wup, wdn, bm):
    S, D = hf.shape
    N = wup.shape[1]
    F = wdn.shape[0]
    return pl.pallas_call(
        _ffn2_kern,
        grid=(S // bm,),
        in_specs=[
            pl.BlockSpec((bm, D), lambda i: (i, 0)),
            pl.BlockSpec((bm, D), lambda i: (i, 0)),
            pl.BlockSpec((D, N), lambda i: (0, 0)),
            pl.BlockSpec((F, D), lambda i: (0, 0)),
        ],
        out_specs=pl.BlockSpec((bm, D), lambda i: (i, 0)),
        out_shape=jax.ShapeDtypeStruct((S, D), F32),
    )(hf, res, wup, wdn)


def _make_sc_router(T, LE, NW=32, L=16):
    tpw = T // NW
    nv = tpw // L
    mesh = plsc.VectorSubcoreMesh(core_axis_name="c", subcore_axis_name="s")

    @functools.partial(
        pl.kernel, mesh=mesh,
        compiler_params=pltpu.CompilerParams(needs_layout_passes=False),
        out_type=jax.ShapeDtypeStruct((LE * T,), F32),
        scratch_types=[
            pltpu.VMEM((T * LE,), F32),
            pltpu.VMEM((LE,), F32),
            pltpu.VMEM((tpw,), jnp.int32),
            pltpu.VMEM((tpw,), jnp.int32),
            pltpu.VMEM((tpw,), F32),
            pltpu.VMEM((tpw,), F32),
            pltpu.VMEM((LE, tpw), F32),
        ],
    )
    def sc_router(tv_hbm, idx0_hbm, idx1_hbm, val0_hbm, val1_hbm, rb_hbm,
                  out_hbm, tv_v, rb_v, i0_v, i1_v, v0_v, v1_v, cb_v):
        wid = lax.axis_index("s") * 2 + lax.axis_index("c")
        base = wid * tpw
        pltpu.sync_copy(tv_hbm, tv_v)
        pltpu.sync_copy(rb_hbm, rb_v)
        pltpu.sync_copy(idx0_hbm.at[pl.ds(base, tpw)], i0_v)
        pltpu.sync_copy(idx1_hbm.at[pl.ds(base, tpw)], i1_v)
        pltpu.sync_copy(val0_hbm.at[pl.ds(base, tpw)], v0_v)
        pltpu.sync_copy(val1_hbm.at[pl.ds(base, tpw)], v1_v)
        for i in range(nv):
            sl = pl.ds(i * L, L)
            t_flat = (lax.iota(jnp.int32, L) + (base + i * L)) * LE
            s_k = []
            idxs = []
            for (iv, vv) in ((i0_v, v0_v), (i1_v, v1_v)):
                ik = iv[sl]
                tvk = plsc.load_gather(tv_v, [t_flat + ik])
                rbk = plsc.load_gather(rb_v, [ik])
                xv = vv[sl] + tvk + rbk
                s_k.append(1.0 / (1.0 + jnp.exp(-xv)))
                idxs.append(ik)
            den = s_k[0] + s_k[1]
            c0 = s_k[0] / den * SCALE
            c1 = s_k[1] / den * SCALE
            zero = jnp.zeros((L,), F32)
            for e in range(LE):
                ce = (jnp.where(idxs[0] == e, c0, zero)
                      + jnp.where(idxs[1] == e, c1, zero))
                cb_v[e, sl] = ce
        for e in range(LE):
            pltpu.sync_copy(cb_v.at[e], out_hbm.at[pl.ds(e * T + base, tpw)])

    return sc_router


def sc_router_call(tv, idx, val, rbias):
    T, LE = tv.shape
    fn = _make_sc_router(T, LE)
    out = fn(tv.reshape(T * LE), idx[:, 0].astype(jnp.int32),
             idx[:, 1].astype(jnp.int32), val[:, 0], val[:, 1], rbias)
    return out.reshape(LE, T).T


def _router_kern(tv_ref, idx_ref, val_ref, rb_ref, comb_ref, *, LE, TK):
    T = tv_ref.shape[0]
    tv = tv_ref[:, :LE]
    idx = idx_ref[...]
    val = val_ref[...]
    rb = rb_ref[...]
    lanes = jax.lax.broadcasted_iota(jnp.int32, (T, LE), 1)
    num = jnp.zeros((T, LE), F32)
    den = jnp.zeros((T, 1), F32)
    for kk in range(TK):
        oh = (idx[:, kk:kk + 1] == lanes).astype(F32)
        tvk = jnp.sum(tv * oh, axis=-1, keepdims=True)
        rbk = jnp.sum(rb * oh, axis=-1, keepdims=True)
        s = jax.nn.sigmoid(val[:, kk:kk + 1] + tvk + rbk)
        num = num + oh * s
        den = den + s
    comb_ref[...] = num / den * SCALE


def router(tv, idx, val, rbias):
    T = tv.shape[0]
    LE = rbias.shape[0]
    TK = idx.shape[1]
    kern = functools.partial(_router_kern, LE=LE, TK=TK)
    return pl.pallas_call(
        kern,
        in_specs=[
            pl.BlockSpec(tv.shape, lambda: (0, 0)),
            pl.BlockSpec((T, TK), lambda: (0, 0)),
            pl.BlockSpec((T, TK), lambda: (0, 0)),
            pl.BlockSpec((1, LE), lambda: (0, 0)),
        ],
        out_specs=pl.BlockSpec((T, LE), lambda: (0, 0)),
        out_shape=jax.ShapeDtypeStruct((T, LE), F32),
    )(tv, idx.astype(jnp.int32), val, rbias.reshape(1, LE))


def _moe_kern(hf_ref, w1_ref, w2_ref, w3_ref, comb_ref, res_ref, o_ref, *, LE):
    e = pl.program_id(0)
    h = hf_ref[...]
    lanes = jax.lax.broadcasted_iota(jnp.int32, (1, LE), 1)
    c = jnp.sum(comb_ref[...] * (lanes == e).astype(F32),
                axis=-1, keepdims=True)
    h1 = _dot(h, w1_ref[0])
    h2 = _dot(h, w2_ref[0])
    hh = h1 * jax.nn.sigmoid(h1) * h2
    yo = _dot(hh, w3_ref[0])

    @pl.when(e == 0)
    def _():
        o_ref[...] = res_ref[...]

    o_ref[...] += c * yo


def moe_experts(hf, w1, w2, w3t, comb, res):
    S, D = hf.shape
    LE = w1.shape[0]
    DE = w1.shape[2]
    kern = functools.partial(_moe_kern, LE=LE)
    return pl.pallas_call(
        kern,
        grid=(LE,),
        in_specs=[
            pl.BlockSpec((S, D), lambda e: (0, 0)),
            pl.BlockSpec((1, D, DE), lambda e: (e, 0, 0)),
            pl.BlockSpec((1, D, DE), lambda e: (e, 0, 0)),
            pl.BlockSpec((1, DE, D), lambda e: (e, 0, 0)),
            pl.BlockSpec((S, LE), lambda e: (0, 0)),
            pl.BlockSpec((S, D), lambda e: (0, 0)),
        ],
        out_specs=pl.BlockSpec((S, D), lambda e: (0, 0)),
        out_shape=jax.ShapeDtypeStruct((S, D), F32),
    )(hf, w1, w2, w3t, comb, res)


def _dense_layer(x, rope_doc, wqkv, wo, wup, wdn, g1, g2, bm, bq):
    cos, sin, cosT, sinT, mask = rope_doc
    qkv = rmsnorm_mm(x, g1, wqkv, bm, out_dtype=BF16)
    xa = attention(qkv, cos, sin, cosT, sinT, mask, bq)
    x2 = mm_add(xa, wo, x, bm)
    y, _ = ffn(x2, g2, wup, wdn, bm)
    return y


def _moe_layer(x, rope_doc, idx, val, wqkv, wo, g1, g2, w1, w2, w3t,
               tkeys_pad, rbias, wup, wdn, bm, bq):
    cos, sin, cosT, sinT, mask = rope_doc
    qkv = rmsnorm_mm(x, g1, wqkv, bm, out_dtype=BF16)
    xa = attention(qkv, cos, sin, cosT, sinT, mask, bq)
    x2, hf = mm_add_norm(xa, wo, x, g2, bm)
    tv = mm_plain(hf, tkeys_pad)
    LE = rbias.shape[0]
    comb = sc_router_call(tv[:, :LE], idx, val, rbias)
    y_sh = ffn_from_hf(hf, x2, wup, wdn, bm)
    return moe_experts(hf, w1, w2, w3t, comb, res=y_sh)


def kernel(x, doc, indices, values, dl_attn_w, dl_attn_o_w, dl_ffn_up_w,
           dl_ffn_down_w, dl_attn_norm, dl_ffn_norm, ml_attn_w, ml_attn_o_w,
           ml_attn_norm, ml_ffn_norm, ml_experts, ml_token_keys,
           ml_router_bias, ml_ffn_up_w, ml_ffn_down_w):
    B, S, D = x.shape
    A = D // H
    bm = min(256, S)
    bq = min(256, S)

    inv = (1.0 / THETA) ** (jnp.arange(0, A, 2, dtype=F32) / A)
    fr = jnp.outer(jnp.arange(S, dtype=F32), inv)
    cos, sin = jnp.cos(fr), jnp.sin(fr)
    doc_flat = doc.reshape(S).astype(jnp.int32)
    doc_start = jnp.searchsorted(doc_flat, doc_flat, side="left")
    start_col = doc_start.reshape(S, 1).astype(jnp.int32)
    mask = build_mask(start_col, S, bq)
    rope_doc = (cos, sin, cos.T, sin.T, mask)

    LE = ml_router_bias.shape[1]
    pad_to = max(128, LE)
    xs = x.reshape(S, D)

    dl_attn_w = dl_attn_w.astype(BF16)
    dl_attn_o_w = dl_attn_o_w.astype(BF16)
    dl_ffn_up_w = dl_ffn_up_w.astype(BF16)
    dl_ffn_down_w = dl_ffn_down_w.astype(BF16)
    ml_attn_w = ml_attn_w.astype(BF16)
    ml_attn_o_w = ml_attn_o_w.astype(BF16)
    ml_ffn_up_w = ml_ffn_up_w.astype(BF16)
    ml_ffn_down_w = ml_ffn_down_w.astype(BF16)
    ml_experts_b = ml_experts.astype(BF16)

    xs = _dense_layer(xs, rope_doc, dl_attn_w[0], dl_attn_o_w[0],
                      dl_ffn_up_w[0], dl_ffn_down_w[0], dl_attn_norm[0],
                      dl_ffn_norm[0], bm, bq)

    L = ml_attn_w.shape[0]
    for j in range(L):
        tkeys_pad = jnp.pad(ml_token_keys[j],
                            ((0, 0), (0, pad_to - LE))).astype(BF16)
        w3t = ml_experts_b[j, 2].transpose(0, 2, 1)
        xs = _moe_layer(xs, rope_doc, indices[j], values[j], ml_attn_w[j],
                        ml_attn_o_w[j], ml_attn_norm[j], ml_ffn_norm[j],
                        ml_experts_b[j, 0], ml_experts_b[j, 1], w3t,
                        tkeys_pad, ml_router_bias[j], ml_ffn_up_w[j],
                        ml_ffn_down_w[j], bm, bq)

    xs = _dense_layer(xs, rope_doc, dl_attn_w[1], dl_attn_o_w[1],
                      dl_ffn_up_w[1], dl_ffn_down_w[1], dl_attn_norm[1],
                      dl_ffn_norm[1], bm, bq)
    return xs.reshape(B, S, D)


# speedup vs baseline: 1.1140x; 1.0602x over previous
import functools
import math

import jax
import jax.numpy as jnp
from jax import lax
from jax.experimental import pallas as pl
from jax.experimental.pallas import tpu as pltpu
from jax.experimental.pallas import tpu_sc as plsc

H = 16
EPS = 1e-5
THETA = 10000.0
SCALE = 1.0
F32 = jnp.float32
NEG = -1e30


BF16 = jnp.bfloat16


def _dot(a, b):
    return jnp.dot(a.astype(BF16), b.astype(BF16),
                   preferred_element_type=F32)

def _rms(x, g):
    return x * jax.lax.rsqrt(jnp.mean(x * x, axis=-1, keepdims=True) + EPS) * g


def _rmsnorm_mm_kern(x_ref, g_ref, w_ref, o_ref):
    h = _rms(x_ref[...], g_ref[...])
    o_ref[...] = _dot(h, w_ref[...]).astype(o_ref.dtype)


def rmsnorm_mm(x, g, w, bm, out_dtype=F32):
    S, D = x.shape
    N = w.shape[1]
    return pl.pallas_call(
        _rmsnorm_mm_kern,
        grid=(S // bm,),
        in_specs=[
            pl.BlockSpec((bm, D), lambda i: (i, 0)),
            pl.BlockSpec((1, D), lambda i: (0, 0)),
            pl.BlockSpec((D, N), lambda i: (0, 0)),
        ],
        out_specs=pl.BlockSpec((bm, N), lambda i: (i, 0)),
        out_shape=jax.ShapeDtypeStruct((S, N), out_dtype),
    )(x, g.reshape(1, D), w)


def _ropek_kern(kt_ref, ct_ref, st_ref, o_ref):
    kt = kt_ref[0].astype(F32)
    half = kt.shape[0] // 2
    ct = ct_ref[...]
    st = st_ref[...]
    o_ref[0] = jnp.concatenate(
        [kt[:half] * ct + kt[half:] * st,
         -kt[:half] * st + kt[half:] * ct], axis=0).astype(o_ref.dtype)


def rope_k(kht, cosT, sinT):
    Hh, A, S = kht.shape
    half = A // 2
    return pl.pallas_call(
        _ropek_kern,
        grid=(Hh,),
        in_specs=[
            pl.BlockSpec((1, A, S), lambda h: (h, 0, 0)),
            pl.BlockSpec((half, S), lambda h: (0, 0)),
            pl.BlockSpec((half, S), lambda h: (0, 0)),
        ],
        out_specs=pl.BlockSpec((1, A, S), lambda h: (h, 0, 0)),
        out_shape=jax.ShapeDtypeStruct((Hh, A, S), BF16),
    )(kht, cosT, sinT)


def _mask_kern(dq_ref, o_ref, *, bq, S):
    i = pl.program_id(0)
    qpos = i * bq + jax.lax.broadcasted_iota(jnp.int32, (bq, 1), 0)
    kpos = jax.lax.broadcasted_iota(jnp.int32, (bq, S), 1)
    ok = (kpos <= qpos) & (kpos >= dq_ref[...])
    o_ref[...] = jnp.where(ok, 0.0, NEG).astype(o_ref.dtype)


def build_mask(doc_start_col, S, bq):
    kern = functools.partial(_mask_kern, bq=bq, S=S)
    return pl.pallas_call(
        kern,
        grid=(S // bq,),
        in_specs=[pl.BlockSpec((bq, 1), lambda i: (i, 0))],
        out_specs=pl.BlockSpec((bq, S), lambda i: (i, 0)),
        out_shape=jax.ShapeDtypeStruct((S, S), F32),
    )(doc_start_col)


def _attn_kern(q_ref, kt_ref, v_ref, cq_ref, sq_ref, mask_ref, o_ref, *, A):
    half = A // 2
    q = q_ref[0].astype(F32)
    q = jnp.concatenate([q[:, :half] * cq_ref[...] + q[:, half:] * sq_ref[...],
                         -q[:, :half] * sq_ref[...] + q[:, half:] * cq_ref[...]],
                        axis=-1) * (1.0 / math.sqrt(A))
    s = jnp.dot(q.astype(BF16), kt_ref[0], preferred_element_type=F32)
    s = s + mask_ref[...]
    m = jnp.max(s, axis=-1, keepdims=True)
    p = jnp.exp(s - m)
    l = jnp.sum(p, axis=-1, keepdims=True)
    o = jnp.dot(p.astype(BF16), v_ref[0], preferred_element_type=F32)
    o_ref[0] = (o / l).astype(o_ref.dtype)


def attention(qkv, cos, sin, cosT, sinT, mask, bq):
    S = qkv.shape[0]
    D = qkv.shape[1] // 3
    A = D // H
    half = A // 2
    nq = S // bq
    qh = qkv[:, :D].reshape(S, H, A).transpose(1, 0, 2)
    kht = rope_k(qkv[:, D:2 * D].reshape(S, H, A).transpose(1, 2, 0),
                 cosT, sinT)
    vh = qkv[:, 2 * D:].reshape(S, H, A).transpose(1, 0, 2)
    kern = functools.partial(_attn_kern, A=A)
    G = 4 if nq % 4 == 0 and (nq // 4) * bq % 128 == 0 else 1
    gsz = nq // G
    outs = []
    for g in range(G):
        goff = g * gsz
        kw = (goff + gsz) * bq
        out = pl.pallas_call(
            kern,
            grid=(gsz, H),
            in_specs=[
                pl.BlockSpec((1, bq, A),
                             lambda i, h, goff=goff: (h, i + goff, 0)),
                pl.BlockSpec((1, A, kw), lambda i, h: (h, 0, 0)),
                pl.BlockSpec((1, kw, A), lambda i, h: (h, 0, 0)),
                pl.BlockSpec((bq, half),
                             lambda i, h, goff=goff: (i + goff, 0)),
                pl.BlockSpec((bq, half),
                             lambda i, h, goff=goff: (i + goff, 0)),
                pl.BlockSpec((bq, kw),
                             lambda i, h, goff=goff: (i + goff, 0)),
            ],
            out_specs=pl.BlockSpec((1, bq, A), lambda i, h: (h, i, 0)),
            out_shape=jax.ShapeDtypeStruct((H, gsz * bq, A), BF16),
        )(qh, kht, vh, cos, sin, mask)
        outs.append(out)
    out = jnp.concatenate(outs, axis=1) if G > 1 else outs[0]
    return out.transpose(1, 0, 2).reshape(S, D)


def _mm_add_kern(a_ref, w_ref, r_ref, o_ref):
    o_ref[...] = _dot(a_ref[...], w_ref[...]) + r_ref[...]


def mm_add(a, w, res, bm):
    S, K = a.shape
    N = w.shape[1]
    return pl.pallas_call(
        _mm_add_kern,
        grid=(S // bm,),
        in_specs=[
            pl.BlockSpec((bm, K), lambda i: (i, 0)),
            pl.BlockSpec((K, N), lambda i: (0, 0)),
            pl.BlockSpec((bm, N), lambda i: (i, 0)),
        ],
        out_specs=pl.BlockSpec((bm, N), lambda i: (i, 0)),
        out_shape=jax.ShapeDtypeStruct((S, N), F32),
    )(a, w, res)


def _ffn_kern(x_ref, g_ref, wu_ref, wd_ref, y_ref, hf_ref):
    x = x_ref[...]
    h = _rms(x, g_ref[...])
    hf_ref[...] = h
    u = _dot(h, wu_ref[...])
    F = wd_ref.shape[0]
    a1 = u[:, :F]
    a2 = u[:, F:]
    gated = a1 * jax.nn.sigmoid(a1) * a2
    y_ref[...] = _dot(gated, wd_ref[...]) + x


def ffn(x2, g, wup, wdn, bm):
    S, D = x2.shape
    N = wup.shape[1]
    F = wdn.shape[0]
    return pl.pallas_call(
        _ffn_kern,
        grid=(S // bm,),
        in_specs=[
            pl.BlockSpec((bm, D), lambda i: (i, 0)),
            pl.BlockSpec((1, D), lambda i: (0, 0)),
            pl.BlockSpec((D, N), lambda i: (0, 0)),
            pl.BlockSpec((F, D), lambda i: (0, 0)),
        ],
        out_specs=[
            pl.BlockSpec((bm, D), lambda i: (i, 0)),
            pl.BlockSpec((bm, D), lambda i: (i, 0)),
        ],
        out_shape=[
            jax.ShapeDtypeStruct((S, D), F32),
            jax.ShapeDtypeStruct((S, D), F32),
        ],
    )(x2, g.reshape(1, D), wup, wdn)


def _mm_kern(a_ref, w_ref, o_ref):
    o_ref[...] = _dot(a_ref[...], w_ref[...])


def mm_plain(a, w):
    S, K = a.shape
    N = w.shape[1]
    return pl.pallas_call(
        _mm_kern,
        in_specs=[pl.BlockSpec((S, K), lambda: (0, 0)),
                  pl.BlockSpec((K, N), lambda: (0, 0))],
        out_specs=pl.BlockSpec((S, N), lambda: (0, 0)),
        out_shape=jax.ShapeDtypeStruct((S, N), F32),
    )(a, w)


def _mm_add_norm_kern(a_ref, w_ref, r_ref, g_ref, o_ref, hf_ref):
    x2 = _dot(a_ref[...], w_ref[...]) + r_ref[...]
    o_ref[...] = x2
    hf_ref[...] = _rms(x2, g_ref[...]).astype(hf_ref.dtype)


def mm_add_norm(a, w, res, g, bm):
    S, K = a.shape
    N = w.shape[1]
    return pl.pallas_call(
        _mm_add_norm_kern,
        grid=(S // bm,),
        in_specs=[
            pl.BlockSpec((bm, K), lambda i: (i, 0)),
            pl.BlockSpec((K, N), lambda i: (0, 0)),
            pl.BlockSpec((bm, N), lambda i: (i, 0)),
            pl.BlockSpec((1, N), lambda i: (0, 0)),
        ],
        out_specs=[
            pl.BlockSpec((bm, N), lambda i: (i, 0)),
            pl.BlockSpec((bm, N), lambda i: (i, 0)),
        ],
        out_shape=[
            jax.ShapeDtypeStruct((S, N), F32),
            jax.ShapeDtypeStruct((S, N), BF16),
        ],
    )(a, w, res, g.reshape(1, N))


def _rmsnorm_kern(x_ref, g_ref, o_ref):
    o_ref[...] = _rms(x_ref[...], g_ref[...]).astype(o_ref.dtype)


def rmsnorm_only(x, g, bm):
    S, D = x.shape
    return pl.pallas_call(
        _rmsnorm_kern,
        grid=(S // bm,),
        in_specs=[
            pl.BlockSpec((bm, D), lambda i: (i, 0)),
            pl.BlockSpec((1, D), lambda i: (0, 0)),
        ],
        out_specs=pl.BlockSpec((bm, D), lambda i: (i, 0)),
        out_shape=jax.ShapeDtypeStruct((S, D), BF16),
    )(x, g.reshape(1, D))


def _ffn2_kern(h_ref, r_ref, wu_ref, wd_ref, y_ref):
    h = h_ref[...]
    u = _dot(h, wu_ref[...])
    F = wd_ref.shape[0]
    a1 = u[:, :F]
    a2 = u[:, F:]
    gated = a1 * jax.nn.sigmoid(a1) * a2
    y_ref[...] = _dot(gated, wd_ref[...]) + r_ref[...]


def ffn_from_hf(hf, res, wup, wdn, bm):
    S, D = hf.shape
    N = wup.shape[1]
    F = wdn.shape[0]
    return pl.pallas_call(
        _ffn2_kern,
        grid=(S // bm,),
        in_specs=[
            pl.BlockSpec((bm, D), lambda i: (i, 0)),
            pl.BlockSpec((bm, D), lambda i: (i, 0)),
            pl.BlockSpec((D, N), lambda i: (0, 0)),
            pl.BlockSpec((F, D), lambda i: (0, 0)),
        ],
        out_specs=pl.BlockSpec((bm, D), lambda i: (i, 0)),
        out_shape=jax.ShapeDtypeStruct((S, D), F32),
    )(hf, res, wup, wdn)


def _make_sc_router(T, LE, NW=32, L=16):
    tpw = T // NW
    nv = tpw // L
    mesh = plsc.VectorSubcoreMesh(core_axis_name="c", subcore_axis_name="s")

    @functools.partial(
        pl.kernel, mesh=mesh,
        compiler_params=pltpu.CompilerParams(needs_layout_passes=False),
        out_type=jax.ShapeDtypeStruct((LE * T,), F32),
        scratch_types=[
            pltpu.VMEM((T * LE,), F32),
            pltpu.VMEM((LE,), F32),
            pltpu.VMEM((tpw,), jnp.int32),
            pltpu.VMEM((tpw,), jnp.int32),
            pltpu.VMEM((tpw,), F32),
            pltpu.VMEM((tpw,), F32),
            pltpu.VMEM((LE, tpw), F32),
        ],
    )
    def sc_router(tv_hbm, idx0_hbm, idx1_hbm, val0_hbm, val1_hbm, rb_hbm,
                  out_hbm, tv_v, rb_v, i0_v, i1_v, v0_v, v1_v, cb_v):
        wid = lax.axis_index("s") * 2 + lax.axis_index("c")
        base = wid * tpw
        pltpu.sync_copy(tv_hbm, tv_v)
        pltpu.sync_copy(rb_hbm, rb_v)
        pltpu.sync_copy(idx0_hbm.at[pl.ds(base, tpw)], i0_v)
        pltpu.sync_copy(idx1_hbm.at[pl.ds(base, tpw)], i1_v)
        pltpu.sync_copy(val0_hbm.at[pl.ds(base, tpw)], v0_v)
        pltpu.sync_copy(val1_hbm.at[pl.ds(base, tpw)], v1_v)
        for i in range(nv):
            sl = pl.ds(i * L, L)
            t_flat = (lax.iota(jnp.int32, L) + (base + i * L)) * LE
            s_k = []
            idxs = []
            for (iv, vv) in ((i0_v, v0_v), (i1_v, v1_v)):
                ik = iv[sl]
                tvk = plsc.load_gather(tv_v, [t_flat + ik])
                rbk = plsc.load_gather(rb_v, [ik])
                xv = vv[sl] + tvk + rbk
                s_k.append(1.0 / (1.0 + jnp.exp(-xv)))
                idxs.append(ik)
            den = s_k[0] + s_k[1]
            c0 = s_k[0] / den * SCALE
            c1 = s_k[1] / den * SCALE
            zero = jnp.zeros((L,), F32)
            for e in range(LE):
                ce = (jnp.where(idxs[0] == e, c0, zero)
                      + jnp.where(idxs[1] == e, c1, zero))
                cb_v[e, sl] = ce
        for e in range(LE):
            pltpu.sync_copy(cb_v.at[e], out_hbm.at[pl.ds(e * T + base, tpw)])

    return sc_router


def sc_router_call(tv, idx, val, rbias):
    T, LE = tv.shape
    fn = _make_sc_router(T, LE)
    out = fn(tv.reshape(T * LE), idx[:, 0].astype(jnp.int32),
             idx[:, 1].astype(jnp.int32), val[:, 0], val[:, 1], rbias)
    return out.reshape(LE, T).T


def _router_kern(tv_ref, idx_ref, val_ref, rb_ref, comb_ref, *, LE, TK):
    T = tv_ref.shape[0]
    tv = tv_ref[:, :LE]
    idx = idx_ref[...]
    val = val_ref[...]
    rb = rb_ref[...]
    lanes = jax.lax.broadcasted_iota(jnp.int32, (T, LE), 1)
    num = jnp.zeros((T, LE), F32)
    den = jnp.zeros((T, 1), F32)
    for kk in range(TK):
        oh = (idx[:, kk:kk + 1] == lanes).astype(F32)
        tvk = jnp.sum(tv * oh, axis=-1, keepdims=True)
        rbk = jnp.sum(rb * oh, axis=-1, keepdims=True)
        s = jax.nn.sigmoid(val[:, kk:kk + 1] + tvk + rbk)
        num = num + oh * s
        den = den + s
    comb_ref[...] = num / den * SCALE


def router(tv, idx, val, rbias):
    T = tv.shape[0]
    LE = rbias.shape[0]
    TK = idx.shape[1]
    kern = functools.partial(_router_kern, LE=LE, TK=TK)
    return pl.pallas_call(
        kern,
        in_specs=[
            pl.BlockSpec(tv.shape, lambda: (0, 0)),
            pl.BlockSpec((T, TK), lambda: (0, 0)),
            pl.BlockSpec((T, TK), lambda: (0, 0)),
            pl.BlockSpec((1, LE), lambda: (0, 0)),
        ],
        out_specs=pl.BlockSpec((T, LE), lambda: (0, 0)),
        out_shape=jax.ShapeDtypeStruct((T, LE), F32),
    )(tv, idx.astype(jnp.int32), val, rbias.reshape(1, LE))


def _moe_kern(hf_ref, w1_ref, w2_ref, w3_ref, comb_ref, res_ref, o_ref, *, LE):
    e = pl.program_id(0)
    h = hf_ref[...]
    lanes = jax.lax.broadcasted_iota(jnp.int32, (1, LE), 1)
    c = jnp.sum(comb_ref[...] * (lanes == e).astype(F32),
                axis=-1, keepdims=True)
    h1 = _dot(h, w1_ref[0])
    h2 = _dot(h, w2_ref[0])
    hh = h1 * jax.nn.sigmoid(h1) * h2
    yo = _dot(hh, w3_ref[0])

    @pl.when(e == 0)
    def _():
        o_ref[...] = res_ref[...]

    o_ref[...] += c * yo


def moe_experts(hf, w1, w2, w3t, comb, res):
    S, D = hf.shape
    LE = w1.shape[0]
    DE = w1.shape[2]
    kern = functools.partial(_moe_kern, LE=LE)
    return pl.pallas_call(
        kern,
        grid=(LE,),
        in_specs=[
            pl.BlockSpec((S, D), lambda e: (0, 0)),
            pl.BlockSpec((1, D, DE), lambda e: (e, 0, 0)),
            pl.BlockSpec((1, D, DE), lambda e: (e, 0, 0)),
            pl.BlockSpec((1, DE, D), lambda e: (e, 0, 0)),
            pl.BlockSpec((S, LE), lambda e: (0, 0)),
            pl.BlockSpec((S, D), lambda e: (0, 0)),
        ],
        out_specs=pl.BlockSpec((S, D), lambda e: (0, 0)),
        out_shape=jax.ShapeDtypeStruct((S, D), F32),
    )(hf, w1, w2, w3t, comb, res)


def _dense_layer(x, rope_doc, wqkv, wo, wup, wdn, g1, g2, bm, bq):
    cos, sin, cosT, sinT, mask = rope_doc
    qkv = rmsnorm_mm(x, g1, wqkv, bm, out_dtype=BF16)
    xa = attention(qkv, cos, sin, cosT, sinT, mask, bq)
    x2 = mm_add(xa, wo, x, bm)
    y, _ = ffn(x2, g2, wup, wdn, bm)
    return y


def _moe_layer(x, rope_doc, idx, val, wqkv, wo, g1, g2, w1, w2, w3t,
               tkeys_pad, rbias, wup, wdn, bm, bq):
    cos, sin, cosT, sinT, mask = rope_doc
    qkv = rmsnorm_mm(x, g1, wqkv, bm, out_dtype=BF16)
    xa = attention(qkv, cos, sin, cosT, sinT, mask, bq)
    x2, hf = mm_add_norm(xa, wo, x, g2, bm)
    tv = mm_plain(hf, tkeys_pad)
    LE = rbias.shape[0]
    comb = sc_router_call(tv[:, :LE], idx, val, rbias)
    y_sh = ffn_from_hf(hf, x2, wup, wdn, bm)
    return moe_experts(hf, w1, w2, w3t, comb, res=y_sh)


def kernel(x, doc, indices, values, dl_attn_w, dl_attn_o_w, dl_ffn_up_w,
           dl_ffn_down_w, dl_attn_norm, dl_ffn_norm, ml_attn_w, ml_attn_o_w,
           ml_attn_norm, ml_ffn_norm, ml_experts, ml_token_keys,
           ml_router_bias, ml_ffn_up_w, ml_ffn_down_w):
    B, S, D = x.shape
    A = D // H
    bm = min(256, S)
    bq = min(256, S)

    inv = (1.0 / THETA) ** (jnp.arange(0, A, 2, dtype=F32) / A)
    fr = jnp.outer(jnp.arange(S, dtype=F32), inv)
    cos, sin = jnp.cos(fr), jnp.sin(fr)
    doc_flat = doc.reshape(S).astype(jnp.int32)
    doc_start = jnp.searchsorted(doc_flat, doc_flat, side="left")
    start_col = doc_start.reshape(S, 1).astype(jnp.int32)
    mask = build_mask(start_col, S, bq)
    rope_doc = (cos, sin, cos.T, sin.T, mask)

    LE = ml_router_bias.shape[1]
    pad_to = max(128, LE)
    xs = x.reshape(S, D)

    dl_attn_w = dl_attn_w.astype(BF16)
    dl_attn_o_w = dl_attn_o_w.astype(BF16)
    dl_ffn_up_w = dl_ffn_up_w.astype(BF16)
    dl_ffn_down_w = dl_ffn_down_w.astype(BF16)
    ml_attn_w = ml_attn_w.astype(BF16)
    ml_attn_o_w = ml_attn_o_w.astype(BF16)
    ml_ffn_up_w = ml_ffn_up_w.astype(BF16)
    ml_ffn_down_w = ml_ffn_down_w.astype(BF16)
    ml_experts_b = ml_experts.astype(BF16)

    xs = _dense_layer(xs, rope_doc, dl_attn_w[0], dl_attn_o_w[0],
                      dl_ffn_up_w[0], dl_ffn_down_w[0], dl_attn_norm[0],
                      dl_ffn_norm[0], bm, bq)

    L = ml_attn_w.shape[0]
    for j in range(L):
        tkeys_pad = jnp.pad(ml_token_keys[j],
                            ((0, 0), (0, pad_to - LE))).astype(BF16)
        w3t = ml_experts_b[j, 2].transpose(0, 2, 1)
        xs = _moe_layer(xs, rope_doc, indices[j], values[j], ml_attn_w[j],
                        ml_attn_o_w[j], ml_attn_norm[j], ml_ffn_norm[j],
                        ml_experts_b[j, 0], ml_experts_b[j, 1], w3t,
                        tkeys_pad, ml_router_bias[j], ml_ffn_up_w[j],
                        ml_ffn_down_w[j], bm, bq)

    xs = _dense_layer(xs, rope_doc, dl_attn_w[1], dl_attn_o_w[1],
                      dl_ffn_up_w[1], dl_ffn_down_w[1], dl_attn_norm[1],
                      dl_ffn_norm[1], bm, bq)
    return xs.reshape(B, S, D)


# speedup vs baseline: 1.2535x; 1.1252x over previous
import functools
import math

import jax
import jax.numpy as jnp
from jax import lax
from jax.experimental import pallas as pl
from jax.experimental.pallas import tpu as pltpu
from jax.experimental.pallas import tpu_sc as plsc

H = 16
EPS = 1e-5
THETA = 10000.0
SCALE = 1.0
F32 = jnp.float32
NEG = -1e30


BF16 = jnp.bfloat16


def _dot(a, b):
    return jnp.dot(a.astype(BF16), b.astype(BF16),
                   preferred_element_type=F32)

def _rms(x, g):
    return x * jax.lax.rsqrt(jnp.mean(x * x, axis=-1, keepdims=True) + EPS) * g


def _rmsnorm_mm_kern(x_ref, g_ref, w_ref, o_ref):
    h = _rms(x_ref[...], g_ref[...])
    o_ref[...] = _dot(h, w_ref[...]).astype(o_ref.dtype)


def rmsnorm_mm(x, g, w, bm, out_dtype=F32):
    S, D = x.shape
    N = w.shape[1]
    return pl.pallas_call(
        _rmsnorm_mm_kern,
        grid=(S // bm,),
        in_specs=[
            pl.BlockSpec((bm, D), lambda i: (i, 0)),
            pl.BlockSpec((1, D), lambda i: (0, 0)),
            pl.BlockSpec((D, N), lambda i: (0, 0)),
        ],
        out_specs=pl.BlockSpec((bm, N), lambda i: (i, 0)),
        out_shape=jax.ShapeDtypeStruct((S, N), out_dtype),
    )(x, g.reshape(1, D), w)


def _ropek_kern(kt_ref, ct_ref, st_ref, o_ref):
    kt = kt_ref[0].astype(F32)
    half = kt.shape[0] // 2
    ct = ct_ref[...]
    st = st_ref[...]
    o_ref[0] = jnp.concatenate(
        [kt[:half] * ct + kt[half:] * st,
         -kt[:half] * st + kt[half:] * ct], axis=0).astype(o_ref.dtype)


def rope_k(kht, cosT, sinT):
    Hh, A, S = kht.shape
    half = A // 2
    return pl.pallas_call(
        _ropek_kern,
        grid=(Hh,),
        in_specs=[
            pl.BlockSpec((1, A, S), lambda h: (h, 0, 0)),
            pl.BlockSpec((half, S), lambda h: (0, 0)),
            pl.BlockSpec((half, S), lambda h: (0, 0)),
        ],
        out_specs=pl.BlockSpec((1, A, S), lambda h: (h, 0, 0)),
        out_shape=jax.ShapeDtypeStruct((Hh, A, S), BF16),
    )(kht, cosT, sinT)


def _mask_kern(dq_ref, o_ref, *, bq, S):
    i = pl.program_id(0)
    qpos = i * bq + jax.lax.broadcasted_iota(jnp.int32, (bq, 1), 0)
    kpos = jax.lax.broadcasted_iota(jnp.int32, (bq, S), 1)
    ok = (kpos <= qpos) & (kpos >= dq_ref[...])
    o_ref[...] = jnp.where(ok, 0.0, NEG).astype(o_ref.dtype)


def build_mask(doc_start_col, S, bq):
    kern = functools.partial(_mask_kern, bq=bq, S=S)
    return pl.pallas_call(
        kern,
        grid=(S // bq,),
        in_specs=[pl.BlockSpec((bq, 1), lambda i: (i, 0))],
        out_specs=pl.BlockSpec((bq, S), lambda i: (i, 0)),
        out_shape=jax.ShapeDtypeStruct((S, S), F32),
    )(doc_start_col)


def _attn_kern(q_ref, kt_ref, v_ref, cq_ref, sq_ref, mask_ref, o_ref,
               *, A, HB):
    half = A // 2
    cq = cq_ref[...]
    sq = sq_ref[...]
    maskv = mask_ref[...]
    for j in range(HB):
        q = q_ref[j].astype(F32)
        q = jnp.concatenate([q[:, :half] * cq + q[:, half:] * sq,
                             -q[:, :half] * sq + q[:, half:] * cq],
                            axis=-1) * (1.0 / math.sqrt(A))
        s = jnp.dot(q.astype(BF16), kt_ref[j], preferred_element_type=F32)
        s = s + maskv
        m = jnp.max(s, axis=-1, keepdims=True)
        p = jnp.exp(s - m)
        l = jnp.sum(p, axis=-1, keepdims=True)
        o = jnp.dot(p.astype(BF16), v_ref[j], preferred_element_type=F32)
        o_ref[j] = (o / l).astype(o_ref.dtype)


def attention(qkv, cos, sin, cosT, sinT, mask, bq):
    S = qkv.shape[0]
    D = qkv.shape[1] // 3
    A = D // H
    half = A // 2
    nq = S // bq
    qh = qkv[:, :D].reshape(S, H, A).transpose(1, 0, 2)
    kht = rope_k(qkv[:, D:2 * D].reshape(S, H, A).transpose(1, 2, 0),
                 cosT, sinT)
    vh = qkv[:, 2 * D:].reshape(S, H, A).transpose(1, 0, 2)
    HB = 4 if H % 4 == 0 else 1
    kern = functools.partial(_attn_kern, A=A, HB=HB)
    G = 4 if nq % 4 == 0 and (nq // 4) * bq % 128 == 0 else 1
    gsz = nq // G
    outs = []
    for g in range(G):
        goff = g * gsz
        kw = (goff + gsz) * bq
        out = pl.pallas_call(
            kern,
            grid=(gsz, H // HB),
            in_specs=[
                pl.BlockSpec((HB, bq, A),
                             lambda i, h, goff=goff: (h, i + goff, 0)),
                pl.BlockSpec((HB, A, kw), lambda i, h: (h, 0, 0)),
                pl.BlockSpec((HB, kw, A), lambda i, h: (h, 0, 0)),
                pl.BlockSpec((bq, half),
                             lambda i, h, goff=goff: (i + goff, 0)),
                pl.BlockSpec((bq, half),
                             lambda i, h, goff=goff: (i + goff, 0)),
                pl.BlockSpec((bq, kw),
                             lambda i, h, goff=goff: (i + goff, 0)),
            ],
            out_specs=pl.BlockSpec((HB, bq, A), lambda i, h: (h, i, 0)),
            out_shape=jax.ShapeDtypeStruct((H, gsz * bq, A), BF16),
        )(qh, kht, vh, cos, sin, mask)
        outs.append(out)
    out = jnp.concatenate(outs, axis=1) if G > 1 else outs[0]
    return out.transpose(1, 0, 2).reshape(S, D)


def _mm_add_kern(a_ref, w_ref, r_ref, o_ref):
    o_ref[...] = _dot(a_ref[...], w_ref[...]) + r_ref[...]


def mm_add(a, w, res, bm):
    S, K = a.shape
    N = w.shape[1]
    return pl.pallas_call(
        _mm_add_kern,
        grid=(S // bm,),
        in_specs=[
            pl.BlockSpec((bm, K), lambda i: (i, 0)),
            pl.BlockSpec((K, N), lambda i: (0, 0)),
            pl.BlockSpec((bm, N), lambda i: (i, 0)),
        ],
        out_specs=pl.BlockSpec((bm, N), lambda i: (i, 0)),
        out_shape=jax.ShapeDtypeStruct((S, N), F32),
    )(a, w, res)


def _ffn_kern(x_ref, g_ref, wu_ref, wd_ref, y_ref, hf_ref):
    x = x_ref[...]
    h = _rms(x, g_ref[...])
    hf_ref[...] = h
    u = _dot(h, wu_ref[...])
    F = wd_ref.shape[0]
    a1 = u[:, :F]
    a2 = u[:, F:]
    gated = a1 * jax.nn.sigmoid(a1) * a2
    y_ref[...] = _dot(gated, wd_ref[...]) + x


def ffn(x2, g, wup, wdn, bm):
    S, D = x2.shape
    N = wup.shape[1]
    F = wdn.shape[0]
    return pl.pallas_call(
        _ffn_kern,
        grid=(S // bm,),
        in_specs=[
            pl.BlockSpec((bm, D), lambda i: (i, 0)),
            pl.BlockSpec((1, D), lambda i: (0, 0)),
            pl.BlockSpec((D, N), lambda i: (0, 0)),
            pl.BlockSpec((F, D), lambda i: (0, 0)),
        ],
        out_specs=[
            pl.BlockSpec((bm, D), lambda i: (i, 0)),
            pl.BlockSpec((bm, D), lambda i: (i, 0)),
        ],
        out_shape=[
            jax.ShapeDtypeStruct((S, D), F32),
            jax.ShapeDtypeStruct((S, D), F32),
        ],
    )(x2, g.reshape(1, D), wup, wdn)


def _mm_kern(a_ref, w_ref, o_ref):
    o_ref[...] = _dot(a_ref[...], w_ref[...])


def mm_plain(a, w):
    S, K = a.shape
    N = w.shape[1]
    return pl.pallas_call(
        _mm_kern,
        in_specs=[pl.BlockSpec((S, K), lambda: (0, 0)),
                  pl.BlockSpec((K, N), lambda: (0, 0))],
        out_specs=pl.BlockSpec((S, N), lambda: (0, 0)),
        out_shape=jax.ShapeDtypeStruct((S, N), F32),
    )(a, w)


def _mm_add_norm_kern(a_ref, w_ref, r_ref, g_ref, o_ref, hf_ref):
    x2 = _dot(a_ref[...], w_ref[...]) + r_ref[...]
    o_ref[...] = x2
    hf_ref[...] = _rms(x2, g_ref[...]).astype(hf_ref.dtype)


def mm_add_norm(a, w, res, g, bm):
    S, K = a.shape
    N = w.shape[1]
    return pl.pallas_call(
        _mm_add_norm_kern,
        grid=(S // bm,),
        in_specs=[
            pl.BlockSpec((bm, K), lambda i: (i, 0)),
            pl.BlockSpec((K, N), lambda i: (0, 0)),
            pl.BlockSpec((bm, N), lambda i: (i, 0)),
            pl.BlockSpec((1, N), lambda i: (0, 0)),
        ],
        out_specs=[
            pl.BlockSpec((bm, N), lambda i: (i, 0)),
            pl.BlockSpec((bm, N), lambda i: (i, 0)),
        ],
        out_shape=[
            jax.ShapeDtypeStruct((S, N), F32),
            jax.ShapeDtypeStruct((S, N), BF16),
        ],
    )(a, w, res, g.reshape(1, N))


def _rmsnorm_kern(x_ref, g_ref, o_ref):
    o_ref[...] = _rms(x_ref[...], g_ref[...]).astype(o_ref.dtype)


def rmsnorm_only(x, g, bm):
    S, D = x.shape
    return pl.pallas_call(
        _rmsnorm_kern,
        grid=(S // bm,),
        in_specs=[
            pl.BlockSpec((bm, D), lambda i: (i, 0)),
            pl.BlockSpec((1, D), lambda i: (0, 0)),
        ],
        out_specs=pl.BlockSpec((bm, D), lambda i: (i, 0)),
        out_shape=jax.ShapeDtypeStruct((S, D), BF16),
    )(x, g.reshape(1, D))


def _ffn2_kern(h_ref, r_ref, wu_ref, wd_ref, y_ref):
    h = h_ref[...]
    u = _dot(h, wu_ref[...])
    F = wd_ref.shape[0]
    a1 = u[:, :F]
    a2 = u[:, F:]
    gated = a1 * jax.nn.sigmoid(a1) * a2
    y_ref[...] = _dot(gated, wd_ref[...]) + r_ref[...]


def ffn_from_hf(hf, res, wup, wdn, bm):
    S, D = hf.shape
    N = wup.shape[1]
    F = wdn.shape[0]
    return pl.pallas_call(
        _ffn2_kern,
        grid=(S // bm,),
        in_specs=[
            pl.BlockSpec((bm, D), lambda i: (i, 0)),
            pl.BlockSpec((bm, D), lambda i: (i, 0)),
            pl.BlockSpec((D, N), lambda i: (0, 0)),
            pl.BlockSpec((F, D), lambda i: (0, 0)),
        ],
        out_specs=pl.BlockSpec((bm, D), lambda i: (i, 0)),
        out_shape=jax.ShapeDtypeStruct((S, D), F32),
    )(hf, res, wup, wdn)


def _make_sc_router(T, LE, NW=32, L=16):
    tpw = T // NW
    nv = tpw // L
    mesh = plsc.VectorSubcoreMesh(core_axis_name="c", subcore_axis_name="s")

    @functools.partial(
        pl.kernel, mesh=mesh,
        compiler_params=pltpu.CompilerParams(needs_layout_passes=False),
        out_type=jax.ShapeDtypeStruct((LE * T,), F32),
        scratch_types=[
            pltpu.VMEM((T * LE,), F32),
            pltpu.VMEM((LE,), F32),
            pltpu.VMEM((tpw,), jnp.int32),
            pltpu.VMEM((tpw,), jnp.int32),
            pltpu.VMEM((tpw,), F32),
            pltpu.VMEM((tpw,), F32),
            pltpu.VMEM((LE, tpw), F32),
        ],
    )
    def sc_router(tv_hbm, idx0_hbm, idx1_hbm, val0_hbm, val1_hbm, rb_hbm,
                  out_hbm, tv_v, rb_v, i0_v, i1_v, v0_v, v1_v, cb_v):
        wid = lax.axis_index("s") * 2 + lax.axis_index("c")
        base = wid * tpw
        pltpu.sync_copy(tv_hbm, tv_v)
        pltpu.sync_copy(rb_hbm, rb_v)
        pltpu.sync_copy(idx0_hbm.at[pl.ds(base, tpw)], i0_v)
        pltpu.sync_copy(idx1_hbm.at[pl.ds(base, tpw)], i1_v)
        pltpu.sync_copy(val0_hbm.at[pl.ds(base, tpw)], v0_v)
        pltpu.sync_copy(val1_hbm.at[pl.ds(base, tpw)], v1_v)
        for i in range(nv):
            sl = pl.ds(i * L, L)
            t_flat = (lax.iota(jnp.int32, L) + (base + i * L)) * LE
            s_k = []
            idxs = []
            for (iv, vv) in ((i0_v, v0_v), (i1_v, v1_v)):
                ik = iv[sl]
                tvk = plsc.load_gather(tv_v, [t_flat + ik])
                rbk = plsc.load_gather(rb_v, [ik])
                xv = vv[sl] + tvk + rbk
                s_k.append(1.0 / (1.0 + jnp.exp(-xv)))
                idxs.append(ik)
            den = s_k[0] + s_k[1]
            c0 = s_k[0] / den * SCALE
            c1 = s_k[1] / den * SCALE
            zero = jnp.zeros((L,), F32)
            for e in range(LE):
                ce = (jnp.where(idxs[0] == e, c0, zero)
                      + jnp.where(idxs[1] == e, c1, zero))
                cb_v[e, sl] = ce
        for e in range(LE):
            pltpu.sync_copy(cb_v.at[e], out_hbm.at[pl.ds(e * T + base, tpw)])

    return sc_router


def sc_router_call(tv, idx, val, rbias):
    T, LE = tv.shape
    fn = _make_sc_router(T, LE)
    out = fn(tv.reshape(T * LE), idx[:, 0].astype(jnp.int32),
             idx[:, 1].astype(jnp.int32), val[:, 0], val[:, 1], rbias)
    return out.reshape(LE, T).T


def _router_kern(tv_ref, idx_ref, val_ref, rb_ref, comb_ref, *, LE, TK):
    T = tv_ref.shape[0]
    tv = tv_ref[:, :LE]
    idx = idx_ref[...]
    val = val_ref[...]
    rb = rb_ref[...]
    lanes = jax.lax.broadcasted_iota(jnp.int32, (T, LE), 1)
    num = jnp.zeros((T, LE), F32)
    den = jnp.zeros((T, 1), F32)
    for kk in range(TK):
        oh = (idx[:, kk:kk + 1] == lanes).astype(F32)
        tvk = jnp.sum(tv * oh, axis=-1, keepdims=True)
        rbk = jnp.sum(rb * oh, axis=-1, keepdims=True)
        s = jax.nn.sigmoid(val[:, kk:kk + 1] + tvk + rbk)
        num = num + oh * s
        den = den + s
    comb_ref[...] = num / den * SCALE


def router(tv, idx, val, rbias):
    T = tv.shape[0]
    LE = rbias.shape[0]
    TK = idx.shape[1]
    kern = functools.partial(_router_kern, LE=LE, TK=TK)
    return pl.pallas_call(
        kern,
        in_specs=[
            pl.BlockSpec(tv.shape, lambda: (0, 0)),
            pl.BlockSpec((T, TK), lambda: (0, 0)),
            pl.BlockSpec((T, TK), lambda: (0, 0)),
            pl.BlockSpec((1, LE), lambda: (0, 0)),
        ],
        out_specs=pl.BlockSpec((T, LE), lambda: (0, 0)),
        out_shape=jax.ShapeDtypeStruct((T, LE), F32),
    )(tv, idx.astype(jnp.int32), val, rbias.reshape(1, LE))


def _moe_kern(hf_ref, w1_ref, w2_ref, w3_ref, comb_ref, res_ref, o_ref, *, LE):
    e = pl.program_id(0)
    h = hf_ref[...]
    lanes = jax.lax.broadcasted_iota(jnp.int32, (1, LE), 1)
    c = jnp.sum(comb_ref[...] * (lanes == e).astype(F32),
                axis=-1, keepdims=True)
    h1 = _dot(h, w1_ref[0])
    h2 = _dot(h, w2_ref[0])
    hh = h1 * jax.nn.sigmoid(h1) * h2
    yo = _dot(hh, w3_ref[0])

    @pl.when(e == 0)
    def _():
        o_ref[...] = res_ref[...]

    o_ref[...] += c * yo


def moe_experts(hf, w1, w2, w3t, comb, res):
    S, D = hf.shape
    LE = w1.shape[0]
    DE = w1.shape[2]
    kern = functools.partial(_moe_kern, LE=LE)
    return pl.pallas_call(
        kern,
        grid=(LE,),
        in_specs=[
            pl.BlockSpec((S, D), lambda e: (0, 0)),
            pl.BlockSpec((1, D, DE), lambda e: (e, 0, 0)),
            pl.BlockSpec((1, D, DE), lambda e: (e, 0, 0)),
            pl.BlockSpec((1, DE, D), lambda e: (e, 0, 0)),
            pl.BlockSpec((S, LE), lambda e: (0, 0)),
            pl.BlockSpec((S, D), lambda e: (0, 0)),
        ],
        out_specs=pl.BlockSpec((S, D), lambda e: (0, 0)),
        out_shape=jax.ShapeDtypeStruct((S, D), F32),
    )(hf, w1, w2, w3t, comb, res)


def _dense_layer(x, rope_doc, wqkv, wo, wup, wdn, g1, g2, bm, bq):
    cos, sin, cosT, sinT, mask = rope_doc
    qkv = rmsnorm_mm(x, g1, wqkv, bm, out_dtype=BF16)
    xa = attention(qkv, cos, sin, cosT, sinT, mask, bq)
    x2 = mm_add(xa, wo, x, bm)
    y, _ = ffn(x2, g2, wup, wdn, bm)
    return y


def _moe_layer(x, rope_doc, idx, val, wqkv, wo, g1, g2, w1, w2, w3t,
               tkeys_pad, rbias, wup, wdn, bm, bq):
    cos, sin, cosT, sinT, mask = rope_doc
    qkv = rmsnorm_mm(x, g1, wqkv, bm, out_dtype=BF16)
    xa = attention(qkv, cos, sin, cosT, sinT, mask, bq)
    x2, hf = mm_add_norm(xa, wo, x, g2, bm)
    tv = mm_plain(hf, tkeys_pad)
    LE = rbias.shape[0]
    comb = sc_router_call(tv[:, :LE], idx, val, rbias)
    y_sh = ffn_from_hf(hf, x2, wup, wdn, bm)
    return moe_experts(hf, w1, w2, w3t, comb, res=y_sh)


def kernel(x, doc, indices, values, dl_attn_w, dl_attn_o_w, dl_ffn_up_w,
           dl_ffn_down_w, dl_attn_norm, dl_ffn_norm, ml_attn_w, ml_attn_o_w,
           ml_attn_norm, ml_ffn_norm, ml_experts, ml_token_keys,
           ml_router_bias, ml_ffn_up_w, ml_ffn_down_w):
    B, S, D = x.shape
    A = D // H
    bm = min(256, S)
    bq = min(256, S)

    inv = (1.0 / THETA) ** (jnp.arange(0, A, 2, dtype=F32) / A)
    fr = jnp.outer(jnp.arange(S, dtype=F32), inv)
    cos, sin = jnp.cos(fr), jnp.sin(fr)
    doc_flat = doc.reshape(S).astype(jnp.int32)
    doc_start = jnp.searchsorted(doc_flat, doc_flat, side="left")
    start_col = doc_start.reshape(S, 1).astype(jnp.int32)
    mask = build_mask(start_col, S, bq)
    rope_doc = (cos, sin, cos.T, sin.T, mask)

    LE = ml_router_bias.shape[1]
    pad_to = max(128, LE)
    xs = x.reshape(S, D)

    dl_attn_w = dl_attn_w.astype(BF16)
    dl_attn_o_w = dl_attn_o_w.astype(BF16)
    dl_ffn_up_w = dl_ffn_up_w.astype(BF16)
    dl_ffn_down_w = dl_ffn_down_w.astype(BF16)
    ml_attn_w = ml_attn_w.astype(BF16)
    ml_attn_o_w = ml_attn_o_w.astype(BF16)
    ml_ffn_up_w = ml_ffn_up_w.astype(BF16)
    ml_ffn_down_w = ml_ffn_down_w.astype(BF16)
    ml_experts_b = ml_experts.astype(BF16)

    xs = _dense_layer(xs, rope_doc, dl_attn_w[0], dl_attn_o_w[0],
                      dl_ffn_up_w[0], dl_ffn_down_w[0], dl_attn_norm[0],
                      dl_ffn_norm[0], bm, bq)

    L = ml_attn_w.shape[0]
    for j in range(L):
        tkeys_pad = jnp.pad(ml_token_keys[j],
                            ((0, 0), (0, pad_to - LE))).astype(BF16)
        w3t = ml_experts_b[j, 2].transpose(0, 2, 1)
        xs = _moe_layer(xs, rope_doc, indices[j], values[j], ml_attn_w[j],
                        ml_attn_o_w[j], ml_attn_norm[j], ml_ffn_norm[j],
                        ml_experts_b[j, 0], ml_experts_b[j, 1], w3t,
                        tkeys_pad, ml_router_bias[j], ml_ffn_up_w[j],
                        ml_ffn_down_w[j], bm, bq)

    xs = _dense_layer(xs, rope_doc, dl_attn_w[1], dl_attn_o_w[1],
                      dl_ffn_up_w[1], dl_ffn_down_w[1], dl_attn_norm[1],
                      dl_ffn_norm[1], bm, bq)
    return xs.reshape(B, S, D)


# speedup vs baseline: 1.2628x; 1.0074x over previous
import functools
import math

import jax
import jax.numpy as jnp
from jax import lax
from jax.experimental import pallas as pl
from jax.experimental.pallas import tpu as pltpu
from jax.experimental.pallas import tpu_sc as plsc

H = 16
EPS = 1e-5
THETA = 10000.0
SCALE = 1.0
F32 = jnp.float32
NEG = -1e30


BF16 = jnp.bfloat16


def _dot(a, b):
    return jnp.dot(a.astype(BF16), b.astype(BF16),
                   preferred_element_type=F32)

def _rms(x, g):
    return x * jax.lax.rsqrt(jnp.mean(x * x, axis=-1, keepdims=True) + EPS) * g


def _rmsnorm_mm_kern(x_ref, g_ref, w_ref, o_ref):
    h = _rms(x_ref[...], g_ref[...])
    o_ref[...] = _dot(h, w_ref[...]).astype(o_ref.dtype)


def rmsnorm_mm(x, g, w, bm, out_dtype=F32):
    S, D = x.shape
    N = w.shape[1]
    return pl.pallas_call(
        _rmsnorm_mm_kern,
        grid=(S // bm,),
        in_specs=[
            pl.BlockSpec((bm, D), lambda i: (i, 0)),
            pl.BlockSpec((1, D), lambda i: (0, 0)),
            pl.BlockSpec((D, N), lambda i: (0, 0)),
        ],
        out_specs=pl.BlockSpec((bm, N), lambda i: (i, 0)),
        out_shape=jax.ShapeDtypeStruct((S, N), out_dtype),
    )(x, g.reshape(1, D), w)


def _ropek_kern(kt_ref, ct_ref, st_ref, o_ref):
    kt = kt_ref[0].astype(F32)
    half = kt.shape[0] // 2
    ct = ct_ref[...]
    st = st_ref[...]
    o_ref[0] = jnp.concatenate(
        [kt[:half] * ct + kt[half:] * st,
         -kt[:half] * st + kt[half:] * ct], axis=0).astype(o_ref.dtype)


def rope_k(kht, cosT, sinT):
    Hh, A, S = kht.shape
    half = A // 2
    return pl.pallas_call(
        _ropek_kern,
        grid=(Hh,),
        in_specs=[
            pl.BlockSpec((1, A, S), lambda h: (h, 0, 0)),
            pl.BlockSpec((half, S), lambda h: (0, 0)),
            pl.BlockSpec((half, S), lambda h: (0, 0)),
        ],
        out_specs=pl.BlockSpec((1, A, S), lambda h: (h, 0, 0)),
        out_shape=jax.ShapeDtypeStruct((Hh, A, S), BF16),
    )(kht, cosT, sinT)


def _mask_kern(dq_ref, o_ref, *, bq, S):
    i = pl.program_id(0)
    qpos = i * bq + jax.lax.broadcasted_iota(jnp.int32, (bq, 1), 0)
    kpos = jax.lax.broadcasted_iota(jnp.int32, (bq, S), 1)
    ok = (kpos <= qpos) & (kpos >= dq_ref[...])
    o_ref[...] = jnp.where(ok, 0.0, NEG).astype(o_ref.dtype)


def build_mask(doc_start_col, S, bq):
    kern = functools.partial(_mask_kern, bq=bq, S=S)
    return pl.pallas_call(
        kern,
        grid=(S // bq,),
        in_specs=[pl.BlockSpec((bq, 1), lambda i: (i, 0))],
        out_specs=pl.BlockSpec((bq, S), lambda i: (i, 0)),
        out_shape=jax.ShapeDtypeStruct((S, S), F32),
    )(doc_start_col)


def _attn_kern(q_ref, kt_ref, v_ref, cq_ref, sq_ref, mask_ref, o_ref,
               *, A, HB):
    half = A // 2
    cq = cq_ref[...]
    sq = sq_ref[...]
    maskv = mask_ref[...]
    for j in range(HB):
        q = q_ref[j].astype(F32)
        q = jnp.concatenate([q[:, :half] * cq + q[:, half:] * sq,
                             -q[:, :half] * sq + q[:, half:] * cq],
                            axis=-1) * (1.0 / math.sqrt(A))
        s = jnp.dot(q.astype(BF16), kt_ref[j], preferred_element_type=F32)
        s = s + maskv
        m = jnp.max(s, axis=-1, keepdims=True)
        p = jnp.exp(s - m)
        l = jnp.sum(p, axis=-1, keepdims=True)
        o = jnp.dot(p.astype(BF16), v_ref[j], preferred_element_type=F32)
        o_ref[j] = (o / l).astype(o_ref.dtype)


def attention(qkv, cos, sin, cosT, sinT, mask, bq):
    S = qkv.shape[0]
    D = qkv.shape[1] // 3
    A = D // H
    half = A // 2
    nq = S // bq
    qh = qkv[:, :D].reshape(S, H, A).transpose(1, 0, 2)
    kht = rope_k(qkv[:, D:2 * D].reshape(S, H, A).transpose(1, 2, 0),
                 cosT, sinT)
    vh = qkv[:, 2 * D:].reshape(S, H, A).transpose(1, 0, 2)
    HB = 8 if H % 8 == 0 else 1
    kern = functools.partial(_attn_kern, A=A, HB=HB)
    G = 4 if nq % 4 == 0 and (nq // 4) * bq % 128 == 0 else 1
    gsz = nq // G
    outs = []
    for g in range(G):
        goff = g * gsz
        kw = (goff + gsz) * bq
        out = pl.pallas_call(
            kern,
            grid=(gsz, H // HB),
            in_specs=[
                pl.BlockSpec((HB, bq, A),
                             lambda i, h, goff=goff: (h, i + goff, 0)),
                pl.BlockSpec((HB, A, kw), lambda i, h: (h, 0, 0)),
                pl.BlockSpec((HB, kw, A), lambda i, h: (h, 0, 0)),
                pl.BlockSpec((bq, half),
                             lambda i, h, goff=goff: (i + goff, 0)),
                pl.BlockSpec((bq, half),
                             lambda i, h, goff=goff: (i + goff, 0)),
                pl.BlockSpec((bq, kw),
                             lambda i, h, goff=goff: (i + goff, 0)),
            ],
            out_specs=pl.BlockSpec((HB, bq, A), lambda i, h: (h, i, 0)),
            out_shape=jax.ShapeDtypeStruct((H, gsz * bq, A), BF16),
        )(qh, kht, vh, cos, sin, mask)
        outs.append(out)
    out = jnp.concatenate(outs, axis=1) if G > 1 else outs[0]
    return out.transpose(1, 0, 2).reshape(S, D)


def _mm_add_kern(a_ref, w_ref, r_ref, o_ref):
    o_ref[...] = _dot(a_ref[...], w_ref[...]) + r_ref[...]


def mm_add(a, w, res, bm):
    S, K = a.shape
    N = w.shape[1]
    return pl.pallas_call(
        _mm_add_kern,
        grid=(S // bm,),
        in_specs=[
            pl.BlockSpec((bm, K), lambda i: (i, 0)),
            pl.BlockSpec((K, N), lambda i: (0, 0)),
            pl.BlockSpec((bm, N), lambda i: (i, 0)),
        ],
        out_specs=pl.BlockSpec((bm, N), lambda i: (i, 0)),
        out_shape=jax.ShapeDtypeStruct((S, N), F32),
    )(a, w, res)


def _ffn_kern(x_ref, g_ref, wu_ref, wd_ref, y_ref, hf_ref):
    x = x_ref[...]
    h = _rms(x, g_ref[...])
    hf_ref[...] = h
    u = _dot(h, wu_ref[...])
    F = wd_ref.shape[0]
    a1 = u[:, :F]
    a2 = u[:, F:]
    gated = a1 * jax.nn.sigmoid(a1) * a2
    y_ref[...] = _dot(gated, wd_ref[...]) + x


def ffn(x2, g, wup, wdn, bm):
    S, D = x2.shape
    N = wup.shape[1]
    F = wdn.shape[0]
    return pl.pallas_call(
        _ffn_kern,
        grid=(S // bm,),
        in_specs=[
            pl.BlockSpec((bm, D), lambda i: (i, 0)),
            pl.BlockSpec((1, D), lambda i: (0, 0)),
            pl.BlockSpec((D, N), lambda i: (0, 0)),
            pl.BlockSpec((F, D), lambda i: (0, 0)),
        ],
        out_specs=[
            pl.BlockSpec((bm, D), lambda i: (i, 0)),
            pl.BlockSpec((bm, D), lambda i: (i, 0)),
        ],
        out_shape=[
            jax.ShapeDtypeStruct((S, D), F32),
            jax.ShapeDtypeStruct((S, D), F32),
        ],
    )(x2, g.reshape(1, D), wup, wdn)


def _mm_kern(a_ref, w_ref, o_ref):
    o_ref[...] = _dot(a_ref[...], w_ref[...])


def mm_plain(a, w):
    S, K = a.shape
    N = w.shape[1]
    return pl.pallas_call(
        _mm_kern,
        in_specs=[pl.BlockSpec((S, K), lambda: (0, 0)),
                  pl.BlockSpec((K, N), lambda: (0, 0))],
        out_specs=pl.BlockSpec((S, N), lambda: (0, 0)),
        out_shape=jax.ShapeDtypeStruct((S, N), F32),
    )(a, w)


def _mm_add_norm_kern(a_ref, w_ref, r_ref, g_ref, o_ref, hf_ref):
    x2 = _dot(a_ref[...], w_ref[...]) + r_ref[...]
    o_ref[...] = x2
    hf_ref[...] = _rms(x2, g_ref[...]).astype(hf_ref.dtype)


def mm_add_norm(a, w, res, g, bm):
    S, K = a.shape
    N = w.shape[1]
    return pl.pallas_call(
        _mm_add_norm_kern,
        grid=(S // bm,),
        in_specs=[
            pl.BlockSpec((bm, K), lambda i: (i, 0)),
            pl.BlockSpec((K, N), lambda i: (0, 0)),
            pl.BlockSpec((bm, N), lambda i: (i, 0)),
            pl.BlockSpec((1, N), lambda i: (0, 0)),
        ],
        out_specs=[
            pl.BlockSpec((bm, N), lambda i: (i, 0)),
            pl.BlockSpec((bm, N), lambda i: (i, 0)),
        ],
        out_shape=[
            jax.ShapeDtypeStruct((S, N), F32),
            jax.ShapeDtypeStruct((S, N), BF16),
        ],
    )(a, w, res, g.reshape(1, N))


def _rmsnorm_kern(x_ref, g_ref, o_ref):
    o_ref[...] = _rms(x_ref[...], g_ref[...]).astype(o_ref.dtype)


def rmsnorm_only(x, g, bm):
    S, D = x.shape
    return pl.pallas_call(
        _rmsnorm_kern,
        grid=(S // bm,),
        in_specs=[
            pl.BlockSpec((bm, D), lambda i: (i, 0)),
            pl.BlockSpec((1, D), lambda i: (0, 0)),
        ],
        out_specs=pl.BlockSpec((bm, D), lambda i: (i, 0)),
        out_shape=jax.ShapeDtypeStruct((S, D), BF16),
    )(x, g.reshape(1, D))


def _ffn2_kern(h_ref, r_ref, wu_ref, wd_ref, y_ref):
    h = h_ref[...]
    u = _dot(h, wu_ref[...])
    F = wd_ref.shape[0]
    a1 = u[:, :F]
    a2 = u[:, F:]
    gated = a1 * jax.nn.sigmoid(a1) * a2
    y_ref[...] = _dot(gated, wd_ref[...]) + r_ref[...]


def ffn_from_hf(hf, res, wup, wdn, bm):
    S, D = hf.shape
    N = wup.shape[1]
    F = wdn.shape[0]
    return pl.pallas_call(
        _ffn2_kern,
        grid=(S // bm,),
        in_specs=[
            pl.BlockSpec((bm, D), lambda i: (i, 0)),
            pl.BlockSpec((bm, D), lambda i: (i, 0)),
            pl.BlockSpec((D, N), lambda i: (0, 0)),
            pl.BlockSpec((F, D), lambda i: (0, 0)),
        ],
        out_specs=pl.BlockSpec((bm, D), lambda i: (i, 0)),
        out_shape=jax.ShapeDtypeStruct((S, D), F32),
    )(hf, res, wup, wdn)


def _make_sc_router(T, LE, NW=32, L=16):
    tpw = T // NW
    nv = tpw // L
    mesh = plsc.VectorSubcoreMesh(core_axis_name="c", subcore_axis_name="s")

    @functools.partial(
        pl.kernel, mesh=mesh,
        compiler_params=pltpu.CompilerParams(needs_layout_passes=False),
        out_type=jax.ShapeDtypeStruct((LE * T,), F32),
        scratch_types=[
            pltpu.VMEM((T * LE,), F32),
            pltpu.VMEM((LE,), F32),
            pltpu.VMEM((tpw,), jnp.int32),
            pltpu.VMEM((tpw,), jnp.int32),
            pltpu.VMEM((tpw,), F32),
            pltpu.VMEM((tpw,), F32),
            pltpu.VMEM((LE, tpw), F32),
        ],
    )
    def sc_router(tv_hbm, idx0_hbm, idx1_hbm, val0_hbm, val1_hbm, rb_hbm,
                  out_hbm, tv_v, rb_v, i0_v, i1_v, v0_v, v1_v, cb_v):
        wid = lax.axis_index("s") * 2 + lax.axis_index("c")
        base = wid * tpw
        pltpu.sync_copy(tv_hbm, tv_v)
        pltpu.sync_copy(rb_hbm, rb_v)
        pltpu.sync_copy(idx0_hbm.at[pl.ds(base, tpw)], i0_v)
        pltpu.sync_copy(idx1_hbm.at[pl.ds(base, tpw)], i1_v)
        pltpu.sync_copy(val0_hbm.at[pl.ds(base, tpw)], v0_v)
        pltpu.sync_copy(val1_hbm.at[pl.ds(base, tpw)], v1_v)
        for i in range(nv):
            sl = pl.ds(i * L, L)
            t_flat = (lax.iota(jnp.int32, L) + (base + i * L)) * LE
            s_k = []
            idxs = []
            for (iv, vv) in ((i0_v, v0_v), (i1_v, v1_v)):
                ik = iv[sl]
                tvk = plsc.load_gather(tv_v, [t_flat + ik])
                rbk = plsc.load_gather(rb_v, [ik])
                xv = vv[sl] + tvk + rbk
                s_k.append(1.0 / (1.0 + jnp.exp(-xv)))
                idxs.append(ik)
            den = s_k[0] + s_k[1]
            c0 = s_k[0] / den * SCALE
            c1 = s_k[1] / den * SCALE
            zero = jnp.zeros((L,), F32)
            for e in range(LE):
                ce = (jnp.where(idxs[0] == e, c0, zero)
                      + jnp.where(idxs[1] == e, c1, zero))
                cb_v[e, sl] = ce
        for e in range(LE):
            pltpu.sync_copy(cb_v.at[e], out_hbm.at[pl.ds(e * T + base, tpw)])

    return sc_router


def sc_router_call(tv, idx, val, rbias):
    T, LE = tv.shape
    fn = _make_sc_router(T, LE)
    out = fn(tv.reshape(T * LE), idx[:, 0].astype(jnp.int32),
             idx[:, 1].astype(jnp.int32), val[:, 0], val[:, 1], rbias)
    return out.reshape(LE, T).T


def _router_kern(tv_ref, idx_ref, val_ref, rb_ref, comb_ref, *, LE, TK):
    T = tv_ref.shape[0]
    tv = tv_ref[:, :LE]
    idx = idx_ref[...]
    val = val_ref[...]
    rb = rb_ref[...]
    lanes = jax.lax.broadcasted_iota(jnp.int32, (T, LE), 1)
    num = jnp.zeros((T, LE), F32)
    den = jnp.zeros((T, 1), F32)
    for kk in range(TK):
        oh = (idx[:, kk:kk + 1] == lanes).astype(F32)
        tvk = jnp.sum(tv * oh, axis=-1, keepdims=True)
        rbk = jnp.sum(rb * oh, axis=-1, keepdims=True)
        s = jax.nn.sigmoid(val[:, kk:kk + 1] + tvk + rbk)
        num = num + oh * s
        den = den + s
    comb_ref[...] = num / den * SCALE


def router(tv, idx, val, rbias):
    T = tv.shape[0]
    LE = rbias.shape[0]
    TK = idx.shape[1]
    kern = functools.partial(_router_kern, LE=LE, TK=TK)
    return pl.pallas_call(
        kern,
        in_specs=[
            pl.BlockSpec(tv.shape, lambda: (0, 0)),
            pl.BlockSpec((T, TK), lambda: (0, 0)),
            pl.BlockSpec((T, TK), lambda: (0, 0)),
            pl.BlockSpec((1, LE), lambda: (0, 0)),
        ],
        out_specs=pl.BlockSpec((T, LE), lambda: (0, 0)),
        out_shape=jax.ShapeDtypeStruct((T, LE), F32),
    )(tv, idx.astype(jnp.int32), val, rbias.reshape(1, LE))


def _moe_kern(hf_ref, w1_ref, w2_ref, w3_ref, comb_ref, res_ref, o_ref, *, LE):
    e = pl.program_id(0)
    h = hf_ref[...]
    lanes = jax.lax.broadcasted_iota(jnp.int32, (1, LE), 1)
    c = jnp.sum(comb_ref[...] * (lanes == e).astype(F32),
                axis=-1, keepdims=True)
    h1 = _dot(h, w1_ref[0])
    h2 = _dot(h, w2_ref[0])
    hh = h1 * jax.nn.sigmoid(h1) * h2
    yo = _dot(hh, w3_ref[0])

    @pl.when(e == 0)
    def _():
        o_ref[...] = res_ref[...]

    o_ref[...] += c * yo


def moe_experts(hf, w1, w2, w3t, comb, res):
    S, D = hf.shape
    LE = w1.shape[0]
    DE = w1.shape[2]
    kern = functools.partial(_moe_kern, LE=LE)
    return pl.pallas_call(
        kern,
        grid=(LE,),
        in_specs=[
            pl.BlockSpec((S, D), lambda e: (0, 0)),
            pl.BlockSpec((1, D, DE), lambda e: (e, 0, 0)),
            pl.BlockSpec((1, D, DE), lambda e: (e, 0, 0)),
            pl.BlockSpec((1, DE, D), lambda e: (e, 0, 0)),
            pl.BlockSpec((S, LE), lambda e: (0, 0)),
            pl.BlockSpec((S, D), lambda e: (0, 0)),
        ],
        out_specs=pl.BlockSpec((S, D), lambda e: (0, 0)),
        out_shape=jax.ShapeDtypeStruct((S, D), F32),
    )(hf, w1, w2, w3t, comb, res)


def _dense_layer(x, rope_doc, wqkv, wo, wup, wdn, g1, g2, bm, bq):
    cos, sin, cosT, sinT, mask = rope_doc
    qkv = rmsnorm_mm(x, g1, wqkv, bm, out_dtype=BF16)
    xa = attention(qkv, cos, sin, cosT, sinT, mask, bq)
    x2 = mm_add(xa, wo, x, bm)
    y, _ = ffn(x2, g2, wup, wdn, bm)
    return y


def _moe_layer(x, rope_doc, idx, val, wqkv, wo, g1, g2, w1, w2, w3t,
               tkeys_pad, rbias, wup, wdn, bm, bq):
    cos, sin, cosT, sinT, mask = rope_doc
    qkv = rmsnorm_mm(x, g1, wqkv, bm, out_dtype=BF16)
    xa = attention(qkv, cos, sin, cosT, sinT, mask, bq)
    x2, hf = mm_add_norm(xa, wo, x, g2, bm)
    tv = mm_plain(hf, tkeys_pad)
    LE = rbias.shape[0]
    comb = sc_router_call(tv[:, :LE], idx, val, rbias)
    y_sh = ffn_from_hf(hf, x2, wup, wdn, bm)
    return moe_experts(hf, w1, w2, w3t, comb, res=y_sh)


def kernel(x, doc, indices, values, dl_attn_w, dl_attn_o_w, dl_ffn_up_w,
           dl_ffn_down_w, dl_attn_norm, dl_ffn_norm, ml_attn_w, ml_attn_o_w,
           ml_attn_norm, ml_ffn_norm, ml_experts, ml_token_keys,
           ml_router_bias, ml_ffn_up_w, ml_ffn_down_w):
    B, S, D = x.shape
    A = D // H
    bm = min(256, S)
    bq = min(256, S)

    inv = (1.0 / THETA) ** (jnp.arange(0, A, 2, dtype=F32) / A)
    fr = jnp.outer(jnp.arange(S, dtype=F32), inv)
    cos, sin = jnp.cos(fr), jnp.sin(fr)
    doc_flat = doc.reshape(S).astype(jnp.int32)
    doc_start = jnp.searchsorted(doc_flat, doc_flat, side="left")
    start_col = doc_start.reshape(S, 1).astype(jnp.int32)
    mask = build_mask(start_col, S, bq)
    rope_doc = (cos, sin, cos.T, sin.T, mask)

    LE = ml_router_bias.shape[1]
    pad_to = max(128, LE)
    xs = x.reshape(S, D)

    dl_attn_w = dl_attn_w.astype(BF16)
    dl_attn_o_w = dl_attn_o_w.astype(BF16)
    dl_ffn_up_w = dl_ffn_up_w.astype(BF16)
    dl_ffn_down_w = dl_ffn_down_w.astype(BF16)
    ml_attn_w = ml_attn_w.astype(BF16)
    ml_attn_o_w = ml_attn_o_w.astype(BF16)
    ml_ffn_up_w = ml_ffn_up_w.astype(BF16)
    ml_ffn_down_w = ml_ffn_down_w.astype(BF16)
    ml_experts_b = ml_experts.astype(BF16)

    xs = _dense_layer(xs, rope_doc, dl_attn_w[0], dl_attn_o_w[0],
                      dl_ffn_up_w[0], dl_ffn_down_w[0], dl_attn_norm[0],
                      dl_ffn_norm[0], bm, bq)

    L = ml_attn_w.shape[0]
    for j in range(L):
        tkeys_pad = jnp.pad(ml_token_keys[j],
                            ((0, 0), (0, pad_to - LE))).astype(BF16)
        w3t = ml_experts_b[j, 2].transpose(0, 2, 1)
        xs = _moe_layer(xs, rope_doc, indices[j], values[j], ml_attn_w[j],
                        ml_attn_o_w[j], ml_attn_norm[j], ml_ffn_norm[j],
                        ml_experts_b[j, 0], ml_experts_b[j, 1], w3t,
                        tkeys_pad, ml_router_bias[j], ml_ffn_up_w[j],
                        ml_ffn_down_w[j], bm, bq)

    xs = _dense_layer(xs, rope_doc, dl_attn_w[1], dl_attn_o_w[1],
                      dl_ffn_up_w[1], dl_ffn_down_w[1], dl_attn_norm[1],
                      dl_ffn_norm[1], bm, bq)
    return xs.reshape(B, S, D)


# speedup vs baseline: 1.2707x; 1.0063x over previous
import functools
import math

import jax
import jax.numpy as jnp
from jax import lax
from jax.experimental import pallas as pl
from jax.experimental.pallas import tpu as pltpu
from jax.experimental.pallas import tpu_sc as plsc

H = 16
EPS = 1e-5
THETA = 10000.0
SCALE = 1.0
F32 = jnp.float32
NEG = -1e30


BF16 = jnp.bfloat16


def _dot(a, b):
    return jnp.dot(a.astype(BF16), b.astype(BF16),
                   preferred_element_type=F32)

def _rms(x, g):
    return x * jax.lax.rsqrt(jnp.mean(x * x, axis=-1, keepdims=True) + EPS) * g


def _rmsnorm_mm_kern(x_ref, g_ref, w_ref, o_ref):
    h = _rms(x_ref[...], g_ref[...])
    o_ref[...] = _dot(h, w_ref[...]).astype(o_ref.dtype)


def rmsnorm_mm(x, g, w, bm, out_dtype=F32):
    S, D = x.shape
    N = w.shape[1]
    return pl.pallas_call(
        _rmsnorm_mm_kern,
        grid=(S // bm,),
        in_specs=[
            pl.BlockSpec((bm, D), lambda i: (i, 0)),
            pl.BlockSpec((1, D), lambda i: (0, 0)),
            pl.BlockSpec((D, N), lambda i: (0, 0)),
        ],
        out_specs=pl.BlockSpec((bm, N), lambda i: (i, 0)),
        out_shape=jax.ShapeDtypeStruct((S, N), out_dtype),
    )(x, g.reshape(1, D), w)


def _ropek_kern(kt_ref, ct_ref, st_ref, o_ref):
    kt = kt_ref[0].astype(F32)
    half = kt.shape[0] // 2
    ct = ct_ref[...]
    st = st_ref[...]
    o_ref[0] = jnp.concatenate(
        [kt[:half] * ct + kt[half:] * st,
         -kt[:half] * st + kt[half:] * ct], axis=0).astype(o_ref.dtype)


def rope_k(kht, cosT, sinT):
    Hh, A, S = kht.shape
    half = A // 2
    return pl.pallas_call(
        _ropek_kern,
        grid=(Hh,),
        in_specs=[
            pl.BlockSpec((1, A, S), lambda h: (h, 0, 0)),
            pl.BlockSpec((half, S), lambda h: (0, 0)),
            pl.BlockSpec((half, S), lambda h: (0, 0)),
        ],
        out_specs=pl.BlockSpec((1, A, S), lambda h: (h, 0, 0)),
        out_shape=jax.ShapeDtypeStruct((Hh, A, S), BF16),
    )(kht, cosT, sinT)


def _mask_kern(dq_ref, o_ref, *, bq, S):
    i = pl.program_id(0)
    qpos = i * bq + jax.lax.broadcasted_iota(jnp.int32, (bq, 1), 0)
    kpos = jax.lax.broadcasted_iota(jnp.int32, (bq, S), 1)
    ok = (kpos <= qpos) & (kpos >= dq_ref[...])
    o_ref[...] = jnp.where(ok, 0.0, NEG).astype(o_ref.dtype)


def build_mask(doc_start_col, S, bq):
    kern = functools.partial(_mask_kern, bq=bq, S=S)
    return pl.pallas_call(
        kern,
        grid=(S // bq,),
        in_specs=[pl.BlockSpec((bq, 1), lambda i: (i, 0))],
        out_specs=pl.BlockSpec((bq, S), lambda i: (i, 0)),
        out_shape=jax.ShapeDtypeStruct((S, S), F32),
    )(doc_start_col)


def _attn_kern(q_ref, kt_ref, v_ref, cq_ref, sq_ref, mask_ref, o_ref,
               *, A, HB):
    half = A // 2
    cq = cq_ref[...]
    sq = sq_ref[...]
    maskv = mask_ref[...]
    for j in range(HB):
        q = q_ref[j].astype(F32)
        q = jnp.concatenate([q[:, :half] * cq + q[:, half:] * sq,
                             -q[:, :half] * sq + q[:, half:] * cq],
                            axis=-1) * (1.0 / math.sqrt(A))
        s = jnp.dot(q.astype(BF16), kt_ref[j], preferred_element_type=F32)
        s = s + maskv
        m = jnp.max(s, axis=-1, keepdims=True)
        p = jnp.exp(s - m)
        l = jnp.sum(p, axis=-1, keepdims=True)
        o = jnp.dot(p.astype(BF16), v_ref[j], preferred_element_type=F32)
        o_ref[j] = (o / l).astype(o_ref.dtype)


def attention(qkv, cos, sin, cosT, sinT, mask, bq):
    S = qkv.shape[0]
    D = qkv.shape[1] // 3
    A = D // H
    half = A // 2
    nq = S // bq
    qh = qkv[:, :D].reshape(S, H, A).transpose(1, 0, 2)
    kht = rope_k(qkv[:, D:2 * D].reshape(S, H, A).transpose(1, 2, 0),
                 cosT, sinT)
    vh = qkv[:, 2 * D:].reshape(S, H, A).transpose(1, 0, 2)
    HB = 8 if H % 8 == 0 else 1
    kern = functools.partial(_attn_kern, A=A, HB=HB)
    G = 4 if nq % 4 == 0 and (nq // 4) * bq % 128 == 0 else 1
    gsz = nq // G
    outs = []
    for g in range(G):
        goff = g * gsz
        kw = (goff + gsz) * bq
        out = pl.pallas_call(
            kern,
            grid=(gsz, H // HB),
            in_specs=[
                pl.BlockSpec((HB, bq, A),
                             lambda i, h, goff=goff: (h, i + goff, 0)),
                pl.BlockSpec((HB, A, kw), lambda i, h: (h, 0, 0)),
                pl.BlockSpec((HB, kw, A), lambda i, h: (h, 0, 0)),
                pl.BlockSpec((bq, half),
                             lambda i, h, goff=goff: (i + goff, 0)),
                pl.BlockSpec((bq, half),
                             lambda i, h, goff=goff: (i + goff, 0)),
                pl.BlockSpec((bq, kw),
                             lambda i, h, goff=goff: (i + goff, 0)),
            ],
            out_specs=pl.BlockSpec((HB, bq, A), lambda i, h: (h, i, 0)),
            out_shape=jax.ShapeDtypeStruct((H, gsz * bq, A), BF16),
        )(qh, kht, vh, cos, sin, mask)
        outs.append(out)
    out = jnp.concatenate(outs, axis=1) if G > 1 else outs[0]
    return out.transpose(1, 0, 2).reshape(S, D)


def _mm_add_kern(a_ref, w_ref, r_ref, o_ref):
    o_ref[...] = _dot(a_ref[...], w_ref[...]) + r_ref[...]


def mm_add(a, w, res, bm):
    S, K = a.shape
    N = w.shape[1]
    return pl.pallas_call(
        _mm_add_kern,
        grid=(S // bm,),
        in_specs=[
            pl.BlockSpec((bm, K), lambda i: (i, 0)),
            pl.BlockSpec((K, N), lambda i: (0, 0)),
            pl.BlockSpec((bm, N), lambda i: (i, 0)),
        ],
        out_specs=pl.BlockSpec((bm, N), lambda i: (i, 0)),
        out_shape=jax.ShapeDtypeStruct((S, N), F32),
    )(a, w, res)


def _ffn_kern(x_ref, g_ref, wu_ref, wd_ref, y_ref, hf_ref):
    x = x_ref[...]
    h = _rms(x, g_ref[...])
    hf_ref[...] = h
    u = _dot(h, wu_ref[...])
    F = wd_ref.shape[0]
    a1 = u[:, :F]
    a2 = u[:, F:]
    gated = a1 * jax.nn.sigmoid(a1) * a2
    y_ref[...] = _dot(gated, wd_ref[...]) + x


def ffn(x2, g, wup, wdn, bm):
    S, D = x2.shape
    N = wup.shape[1]
    F = wdn.shape[0]
    return pl.pallas_call(
        _ffn_kern,
        grid=(S // bm,),
        in_specs=[
            pl.BlockSpec((bm, D), lambda i: (i, 0)),
            pl.BlockSpec((1, D), lambda i: (0, 0)),
            pl.BlockSpec((D, N), lambda i: (0, 0)),
            pl.BlockSpec((F, D), lambda i: (0, 0)),
        ],
        out_specs=[
            pl.BlockSpec((bm, D), lambda i: (i, 0)),
            pl.BlockSpec((bm, D), lambda i: (i, 0)),
        ],
        out_shape=[
            jax.ShapeDtypeStruct((S, D), F32),
            jax.ShapeDtypeStruct((S, D), F32),
        ],
    )(x2, g.reshape(1, D), wup, wdn)


def _mm_kern(a_ref, w_ref, o_ref):
    o_ref[...] = _dot(a_ref[...], w_ref[...])


def mm_plain(a, w):
    S, K = a.shape
    N = w.shape[1]
    return pl.pallas_call(
        _mm_kern,
        in_specs=[pl.BlockSpec((S, K), lambda: (0, 0)),
                  pl.BlockSpec((K, N), lambda: (0, 0))],
        out_specs=pl.BlockSpec((S, N), lambda: (0, 0)),
        out_shape=jax.ShapeDtypeStruct((S, N), F32),
    )(a, w)


def _mm_add_norm_kern(a_ref, w_ref, r_ref, g_ref, o_ref, hf_ref):
    x2 = _dot(a_ref[...], w_ref[...]) + r_ref[...]
    o_ref[...] = x2
    hf_ref[...] = _rms(x2, g_ref[...]).astype(hf_ref.dtype)


def mm_add_norm(a, w, res, g, bm):
    S, K = a.shape
    N = w.shape[1]
    return pl.pallas_call(
        _mm_add_norm_kern,
        grid=(S // bm,),
        in_specs=[
            pl.BlockSpec((bm, K), lambda i: (i, 0)),
            pl.BlockSpec((K, N), lambda i: (0, 0)),
            pl.BlockSpec((bm, N), lambda i: (i, 0)),
            pl.BlockSpec((1, N), lambda i: (0, 0)),
        ],
        out_specs=[
            pl.BlockSpec((bm, N), lambda i: (i, 0)),
            pl.BlockSpec((bm, N), lambda i: (i, 0)),
        ],
        out_shape=[
            jax.ShapeDtypeStruct((S, N), F32),
            jax.ShapeDtypeStruct((S, N), BF16),
        ],
    )(a, w, res, g.reshape(1, N))


def _rmsnorm_kern(x_ref, g_ref, o_ref):
    o_ref[...] = _rms(x_ref[...], g_ref[...]).astype(o_ref.dtype)


def rmsnorm_only(x, g, bm):
    S, D = x.shape
    return pl.pallas_call(
        _rmsnorm_kern,
        grid=(S // bm,),
        in_specs=[
            pl.BlockSpec((bm, D), lambda i: (i, 0)),
            pl.BlockSpec((1, D), lambda i: (0, 0)),
        ],
        out_specs=pl.BlockSpec((bm, D), lambda i: (i, 0)),
        out_shape=jax.ShapeDtypeStruct((S, D), BF16),
    )(x, g.reshape(1, D))


def _ffn2_kern(h_ref, r_ref, wu_ref, wd_ref, y_ref):
    h = h_ref[...]
    u = _dot(h, wu_ref[...])
    F = wd_ref.shape[0]
    a1 = u[:, :F]
    a2 = u[:, F:]
    gated = a1 * jax.nn.sigmoid(a1) * a2
    y_ref[...] = _dot(gated, wd_ref[...]) + r_ref[...]


def ffn_from_hf(hf, res, wup, wdn, bm):
    S, D = hf.shape
    N = wup.shape[1]
    F = wdn.shape[0]
    return pl.pallas_call(
        _ffn2_kern,
        grid=(S // bm,),
        in_specs=[
            pl.BlockSpec((bm, D), lambda i: (i, 0)),
            pl.BlockSpec((bm, D), lambda i: (i, 0)),
            pl.BlockSpec((D, N), lambda i: (0, 0)),
            pl.BlockSpec((F, D), lambda i: (0, 0)),
        ],
        out_specs=pl.BlockSpec((bm, D), lambda i: (i, 0)),
        out_shape=jax.ShapeDtypeStruct((S, D), F32),
    )(hf, res, wup, wdn)


def _make_sc_router(T, LE, NW=32, L=16):
    tpw = T // NW
    nv = tpw // L
    mesh = plsc.VectorSubcoreMesh(core_axis_name="c", subcore_axis_name="s")

    @functools.partial(
        pl.kernel, mesh=mesh,
        compiler_params=pltpu.CompilerParams(needs_layout_passes=False),
        out_type=jax.ShapeDtypeStruct((LE * T,), F32),
        scratch_types=[
            pltpu.VMEM((T * LE,), F32),
            pltpu.VMEM((LE,), F32),
            pltpu.VMEM((tpw,), jnp.int32),
            pltpu.VMEM((tpw,), jnp.int32),
            pltpu.VMEM((tpw,), F32),
            pltpu.VMEM((tpw,), F32),
            pltpu.VMEM((LE, tpw), F32),
        ],
    )
    def sc_router(tv_hbm, idx0_hbm, idx1_hbm, val0_hbm, val1_hbm, rb_hbm,
                  out_hbm, tv_v, rb_v, i0_v, i1_v, v0_v, v1_v, cb_v):
        wid = lax.axis_index("s") * 2 + lax.axis_index("c")
        base = wid * tpw
        pltpu.sync_copy(tv_hbm, tv_v)
        pltpu.sync_copy(rb_hbm, rb_v)
        pltpu.sync_copy(idx0_hbm.at[pl.ds(base, tpw)], i0_v)
        pltpu.sync_copy(idx1_hbm.at[pl.ds(base, tpw)], i1_v)
        pltpu.sync_copy(val0_hbm.at[pl.ds(base, tpw)], v0_v)
        pltpu.sync_copy(val1_hbm.at[pl.ds(base, tpw)], v1_v)
        for i in range(nv):
            sl = pl.ds(i * L, L)
            t_flat = (lax.iota(jnp.int32, L) + (base + i * L)) * LE
            s_k = []
            idxs = []
            for (iv, vv) in ((i0_v, v0_v), (i1_v, v1_v)):
                ik = iv[sl]
                tvk = plsc.load_gather(tv_v, [t_flat + ik])
                rbk = plsc.load_gather(rb_v, [ik])
                xv = vv[sl] + tvk + rbk
                s_k.append(1.0 / (1.0 + jnp.exp(-xv)))
                idxs.append(ik)
            den = s_k[0] + s_k[1]
            c0 = s_k[0] / den * SCALE
            c1 = s_k[1] / den * SCALE
            zero = jnp.zeros((L,), F32)
            for e in range(LE):
                ce = (jnp.where(idxs[0] == e, c0, zero)
                      + jnp.where(idxs[1] == e, c1, zero))
                cb_v[e, sl] = ce
        for e in range(LE):
            pltpu.sync_copy(cb_v.at[e], out_hbm.at[pl.ds(e * T + base, tpw)])

    return sc_router


def sc_router_call(tv, idx, val, rbias):
    T, LE = tv.shape
    fn = _make_sc_router(T, LE)
    out = fn(tv.reshape(T * LE), idx[:, 0].astype(jnp.int32),
             idx[:, 1].astype(jnp.int32), val[:, 0], val[:, 1], rbias)
    return out.reshape(LE, T).T


def _router_kern(tv_ref, idx_ref, val_ref, rb_ref, comb_ref, *, LE, TK):
    T = tv_ref.shape[0]
    tv = tv_ref[:, :LE]
    idx = idx_ref[...]
    val = val_ref[...]
    rb = rb_ref[...]
    lanes = jax.lax.broadcasted_iota(jnp.int32, (T, LE), 1)
    num = jnp.zeros((T, LE), F32)
    den = jnp.zeros((T, 1), F32)
    for kk in range(TK):
        oh = (idx[:, kk:kk + 1] == lanes).astype(F32)
        tvk = jnp.sum(tv * oh, axis=-1, keepdims=True)
        rbk = jnp.sum(rb * oh, axis=-1, keepdims=True)
        s = jax.nn.sigmoid(val[:, kk:kk + 1] + tvk + rbk)
        num = num + oh * s
        den = den + s
    comb_ref[...] = num / den * SCALE


def router(tv, idx, val, rbias):
    T = tv.shape[0]
    LE = rbias.shape[0]
    TK = idx.shape[1]
    kern = functools.partial(_router_kern, LE=LE, TK=TK)
    return pl.pallas_call(
        kern,
        in_specs=[
            pl.BlockSpec(tv.shape, lambda: (0, 0)),
            pl.BlockSpec((T, TK), lambda: (0, 0)),
            pl.BlockSpec((T, TK), lambda: (0, 0)),
            pl.BlockSpec((1, LE), lambda: (0, 0)),
        ],
        out_specs=pl.BlockSpec((T, LE), lambda: (0, 0)),
        out_shape=jax.ShapeDtypeStruct((T, LE), F32),
    )(tv, idx.astype(jnp.int32), val, rbias.reshape(1, LE))


def _moe_kern(hf_ref, w1_ref, w2_ref, w3_ref, comb_ref, res_ref, o_ref, *, LE):
    e = pl.program_id(0)
    h = hf_ref[...]
    lanes = jax.lax.broadcasted_iota(jnp.int32, (1, LE), 1)
    c = jnp.sum(comb_ref[...] * (lanes == e).astype(F32),
                axis=-1, keepdims=True)
    h1 = _dot(h, w1_ref[0])
    h2 = _dot(h, w2_ref[0])
    hh = h1 * jax.nn.sigmoid(h1) * h2
    yo = _dot(hh, w3_ref[0])

    @pl.when(e == 0)
    def _():
        o_ref[...] = res_ref[...]

    o_ref[...] += c * yo


def moe_experts(hf, w1, w2, w3t, comb, res):
    S, D = hf.shape
    LE = w1.shape[0]
    DE = w1.shape[2]
    kern = functools.partial(_moe_kern, LE=LE)
    return pl.pallas_call(
        kern,
        grid=(LE,),
        in_specs=[
            pl.BlockSpec((S, D), lambda e: (0, 0)),
            pl.BlockSpec((1, D, DE), lambda e: (e, 0, 0)),
            pl.BlockSpec((1, D, DE), lambda e: (e, 0, 0)),
            pl.BlockSpec((1, DE, D), lambda e: (e, 0, 0)),
            pl.BlockSpec((S, LE), lambda e: (0, 0)),
            pl.BlockSpec((S, D), lambda e: (0, 0)),
        ],
        out_specs=pl.BlockSpec((S, D), lambda e: (0, 0)),
        out_shape=jax.ShapeDtypeStruct((S, D), F32),
    )(hf, w1, w2, w3t, comb, res)


def _dense_layer(x, rope_doc, wqkv, wo, wup, wdn, g1, g2, bm, bq):
    cos, sin, cosT, sinT, mask = rope_doc
    qkv = rmsnorm_mm(x, g1, wqkv, bm, out_dtype=BF16)
    xa = attention(qkv, cos, sin, cosT, sinT, mask, bq)
    x2 = mm_add(xa, wo, x, bm)
    y, _ = ffn(x2, g2, wup, wdn, bm)
    return y


def _moe_layer(x, rope_doc, idx, val, wqkv, wo, g1, g2, w1, w2, w3t,
               tkeys_pad, rbias, wup, wdn, bm, bq):
    cos, sin, cosT, sinT, mask = rope_doc
    qkv = rmsnorm_mm(x, g1, wqkv, bm, out_dtype=BF16)
    xa = attention(qkv, cos, sin, cosT, sinT, mask, bq)
    x2, hf = mm_add_norm(xa, wo, x, g2, bm)
    tv = mm_plain(hf, tkeys_pad)
    LE = rbias.shape[0]
    comb = sc_router_call(tv[:, :LE], idx, val, rbias)
    y_sh = ffn_from_hf(hf, x2, wup, wdn, bm)
    return moe_experts(hf, w1, w2, w3t, comb, res=y_sh)


def kernel(x, doc, indices, values, dl_attn_w, dl_attn_o_w, dl_ffn_up_w,
           dl_ffn_down_w, dl_attn_norm, dl_ffn_norm, ml_attn_w, ml_attn_o_w,
           ml_attn_norm, ml_ffn_norm, ml_experts, ml_token_keys,
           ml_router_bias, ml_ffn_up_w, ml_ffn_down_w):
    B, S, D = x.shape
    A = D // H
    bm = min(512, S)
    bq = min(256, S)

    inv = (1.0 / THETA) ** (jnp.arange(0, A, 2, dtype=F32) / A)
    fr = jnp.outer(jnp.arange(S, dtype=F32), inv)
    cos, sin = jnp.cos(fr), jnp.sin(fr)
    doc_flat = doc.reshape(S).astype(jnp.int32)
    doc_start = jnp.searchsorted(doc_flat, doc_flat, side="left")
    start_col = doc_start.reshape(S, 1).astype(jnp.int32)
    mask = build_mask(start_col, S, bq)
    rope_doc = (cos, sin, cos.T, sin.T, mask)

    LE = ml_router_bias.shape[1]
    pad_to = max(128, LE)
    xs = x.reshape(S, D)

    dl_attn_w = dl_attn_w.astype(BF16)
    dl_attn_o_w = dl_attn_o_w.astype(BF16)
    dl_ffn_up_w = dl_ffn_up_w.astype(BF16)
    dl_ffn_down_w = dl_ffn_down_w.astype(BF16)
    ml_attn_w = ml_attn_w.astype(BF16)
    ml_attn_o_w = ml_attn_o_w.astype(BF16)
    ml_ffn_up_w = ml_ffn_up_w.astype(BF16)
    ml_ffn_down_w = ml_ffn_down_w.astype(BF16)
    ml_experts_b = ml_experts.astype(BF16)

    xs = _dense_layer(xs, rope_doc, dl_attn_w[0], dl_attn_o_w[0],
                      dl_ffn_up_w[0], dl_ffn_down_w[0], dl_attn_norm[0],
                      dl_ffn_norm[0], bm, bq)

    L = ml_attn_w.shape[0]
    for j in range(L):
        tkeys_pad = jnp.pad(ml_token_keys[j],
                            ((0, 0), (0, pad_to - LE))).astype(BF16)
        w3t = ml_experts_b[j, 2].transpose(0, 2, 1)
        xs = _moe_layer(xs, rope_doc, indices[j], values[j], ml_attn_w[j],
                        ml_attn_o_w[j], ml_attn_norm[j], ml_ffn_norm[j],
                        ml_experts_b[j, 0], ml_experts_b[j, 1], w3t,
                        tkeys_pad, ml_router_bias[j], ml_ffn_up_w[j],
                        ml_ffn_down_w[j], bm, bq)

    xs = _dense_layer(xs, rope_doc, dl_attn_w[1], dl_attn_o_w[1],
                      dl_ffn_up_w[1], dl_ffn_down_w[1], dl_attn_norm[1],
                      dl_ffn_norm[1], bm, bq)
    return xs.reshape(B, S, D)
